# Initial kernel scaffold; baseline (speedup 1.0000x reference)
import jax
import jax.numpy as jnp
from jax.experimental import pallas as pl


def kernel(features_1, features_2, edge_index_1, edge_index_2, W1, W2, ntn_W, ntn_V, ntn_b, u):
    raise NotImplementedError("write your pallas kernel here")


# speedup vs baseline: 11.3216x; 11.3216x over previous
import functools

import jax
import jax.numpy as jnp
import numpy as np
from jax import lax
from jax.experimental import pallas as pl
from jax.experimental.pallas import tpu as pltpu
from jax.experimental.pallas import tpu_sc as plsc

N = 10000
E = 320000
D_IN = 128
H1 = 64
H2 = 32
K_NTN = 16

NN = 2 * N
NP = 20096
PAD_ROW = NN

NC = 2
NS = 16
NW = NC * NS
CH = 128
CPW = 160
EP = NW * CPW * CH
KSUB = 8
ITERS = CPW // KSUB
RPT = NP // NS
KSUB_BY_D = {64: 4, 32: 8}

_f32 = jnp.float32


def _sc_mesh():
    return plsc.VectorSubcoreMesh(core_axis_name="c", subcore_axis_name="s")


_SC_PARAMS = pltpu.CompilerParams(use_tc_tiling_on_sc=False)


@functools.partial(
    pl.kernel,
    out_type=jax.ShapeDtypeStruct((NC, NP, 16), _f32),
    mesh=_sc_mesh(),
    compiler_params=_SC_PARAMS,
    scratch_types=[
        pltpu.VMEM_SHARED((NP, 16), _f32),
        pltpu.VMEM((KSUB, CH), jnp.int32),
        pltpu.VMEM((CH, 16), _f32),
    ],
)
def _sc_degree(dst_hbm, ones_hbm, zeros_hbm, out_hbm, acc, didx, ones_v):
    c = lax.axis_index("c")
    s = lax.axis_index("s")
    wid = s * NC + c
    pltpu.sync_copy(zeros_hbm.at[pl.ds(s * RPT, RPT)], acc.at[pl.ds(s * RPT, RPT)])
    pltpu.sync_copy(ones_hbm, ones_v)
    plsc.subcore_barrier()

    def body(t, carry):
        row0 = wid * CPW + t * KSUB
        pltpu.sync_copy(dst_hbm.at[pl.ds(row0, KSUB)], didx)
        for j in range(KSUB):
            pltpu.sync_copy(ones_v, acc.at[didx.at[j]], add=True)
        return carry

    lax.fori_loop(0, ITERS, body, 0)
    plsc.subcore_barrier()
    pltpu.sync_copy(acc.at[pl.ds(s * RPT, RPT)], out_hbm.at[c, pl.ds(s * RPT, RPT)])


def _make_segsum(d_feat):
    ksub = KSUB_BY_D[d_feat]
    iters = CPW // ksub

    @functools.partial(
        pl.kernel,
        out_type=jax.ShapeDtypeStruct((NC, NP, d_feat), _f32),
        mesh=_sc_mesh(),
        compiler_params=_SC_PARAMS,
        scratch_types=[
            pltpu.VMEM_SHARED((NP, d_feat), _f32),
            pltpu.VMEM((ksub, CH), jnp.int32),
            pltpu.VMEM((ksub, CH), jnp.int32),
            pltpu.VMEM((ksub, CH, d_feat), _f32),
            pltpu.SemaphoreType.DMA,
        ],
    )
    def seg(z_hbm, src_hbm, dst_hbm, zeros_hbm, out_hbm, acc, sidx, didx, rows, sem):
        c = lax.axis_index("c")
        s = lax.axis_index("s")
        wid = s * NC + c
        pltpu.sync_copy(zeros_hbm.at[pl.ds(s * RPT, RPT)], acc.at[pl.ds(s * RPT, RPT)])
        plsc.subcore_barrier()

        def body(t, carry):
            row0 = wid * CPW + t * ksub
            pltpu.sync_copy(src_hbm.at[pl.ds(row0, ksub)], sidx)
            pltpu.sync_copy(dst_hbm.at[pl.ds(row0, ksub)], didx)
            descs = [
                pltpu.async_copy(z_hbm.at[sidx.at[j]], rows.at[j], sem)
                for j in range(ksub)
            ]
            for dsc in descs:
                dsc.wait()
            for j in range(ksub):
                pltpu.sync_copy(rows.at[j], acc.at[didx.at[j]], add=True)
            return carry

        lax.fori_loop(0, iters, body, 0)
        plsc.subcore_barrier()
        pltpu.sync_copy(acc.at[pl.ds(s * RPT, RPT)], out_hbm.at[c, pl.ds(s * RPT, RPT)])

    return seg


_segsum_64 = _make_segsum(H1)
_segsum_32 = _make_segsum(H2)


BM = 1256


def _r_from_deg(d0, d1):
    deg = d0[:, :1] + d1[:, :1]
    return lax.rsqrt(jnp.maximum(deg, 1.0))


def _mm1_body(x_ref, d0_ref, d1_ref, w_ref, o_ref):
    r = _r_from_deg(d0_ref[...], d1_ref[...])
    z = jnp.dot(x_ref[...], w_ref[...], preferred_element_type=_f32)
    o_ref[...] = r * z


def _mm2_body(p0_ref, p1_ref, d0_ref, d1_ref, w_ref, o_ref):
    r = _r_from_deg(d0_ref[...], d1_ref[...])
    h = jnp.maximum(r * (p0_ref[...] + p1_ref[...]), 0.0)
    o_ref[...] = r * jnp.dot(h, w_ref[...], preferred_element_type=_f32)


BP = 1000


def _pool_body(p0_ref, p1_ref, d0_ref, d1_ref, o_ref):
    i = pl.program_id(0)
    r = _r_from_deg(d0_ref[...], d1_ref[...])
    h = jnp.maximum(r * (p0_ref[...] + p1_ref[...]), 0.0)
    colsum = jnp.sum(h, axis=0, keepdims=True) * np.float32(1.0 / N)

    @pl.when(i == 0)
    def _():
        o_ref[...] = jnp.zeros_like(o_ref)

    @pl.when(i < 10)
    def _():
        o_ref[0:1, :] += colsum

    @pl.when(i >= 10)
    def _():
        o_ref[1:2, :] += colsum


def _ntn_body(p_ref, w_ref, v_ref, b_ref, u_ref, o_ref):
    p = p_ref[...]
    h1 = p[0:1, :]
    h2 = p[1:2, :]
    w = w_ref[...]
    t = jnp.sum(w * h2[None, :, :], axis=2)
    bil = jnp.sum(t * h1, axis=1, keepdims=True)
    v = v_ref[...]
    lin = (jnp.sum(v[:, :H2] * h1, axis=1, keepdims=True)
           + jnp.sum(v[:, H2:] * h2, axis=1, keepdims=True))
    scores = jnp.maximum(bil + lin + b_ref[...], 0.0)
    val = jnp.sum(u_ref[...] * scores, keepdims=True)
    o_ref[...] = 1.0 / (1.0 + jnp.exp(-val))


def kernel(features_1, features_2, edge_index_1, edge_index_2,
           W1, W2, ntn_W, ntn_V, ntn_b, u):
    x = jnp.concatenate([features_1, features_2], axis=0)
    x = jnp.pad(x, ((0, NP - NN), (0, 0)))
    src = jnp.concatenate([
        edge_index_1[0], edge_index_2[0] + N,
        jnp.full((EP - 2 * E,), PAD_ROW, jnp.int32),
    ]).reshape(EP // CH, CH)
    dst = jnp.concatenate([
        edge_index_1[1], edge_index_2[1] + N,
        jnp.full((EP - 2 * E,), PAD_ROW, jnp.int32),
    ]).reshape(EP // CH, CH)

    ones16 = jnp.ones((CH, 16), _f32)
    zeros16 = jnp.zeros((NP, 16), _f32)
    zeros64 = jnp.zeros((NP, H1), _f32)
    zeros32 = jnp.zeros((NP, H2), _f32)

    dpart = _sc_degree(dst, ones16, zeros16)
    d0, d1 = dpart[0], dpart[1]

    grid = NP // BM
    z1 = pl.pallas_call(
        _mm1_body,
        grid=(grid,),
        in_specs=[
            pl.BlockSpec((BM, D_IN), lambda i: (i, 0)),
            pl.BlockSpec((BM, 16), lambda i: (i, 0)),
            pl.BlockSpec((BM, 16), lambda i: (i, 0)),
            pl.BlockSpec((D_IN, H1), lambda i: (0, 0)),
        ],
        out_specs=pl.BlockSpec((BM, H1), lambda i: (i, 0)),
        out_shape=jax.ShapeDtypeStruct((NP, H1), _f32),
    )(x, d0, d1, W1)

    p = _segsum_64(z1, src, dst, zeros64)

    z2 = pl.pallas_call(
        _mm2_body,
        grid=(grid,),
        in_specs=[
            pl.BlockSpec((BM, H1), lambda i: (i, 0)),
            pl.BlockSpec((BM, H1), lambda i: (i, 0)),
            pl.BlockSpec((BM, 16), lambda i: (i, 0)),
            pl.BlockSpec((BM, 16), lambda i: (i, 0)),
            pl.BlockSpec((H1, H2), lambda i: (0, 0)),
        ],
        out_specs=pl.BlockSpec((BM, H2), lambda i: (i, 0)),
        out_shape=jax.ShapeDtypeStruct((NP, H2), _f32),
    )(p[0], p[1], d0, d1, W2)

    q = _segsum_32(z2, src, dst, zeros32)

    pooled = pl.pallas_call(
        _pool_body,
        grid=(2 * N // BP,),
        in_specs=[
            pl.BlockSpec((BP, H2), lambda i: (i, 0)),
            pl.BlockSpec((BP, H2), lambda i: (i, 0)),
            pl.BlockSpec((BP, 16), lambda i: (i, 0)),
            pl.BlockSpec((BP, 16), lambda i: (i, 0)),
        ],
        out_specs=pl.BlockSpec((2, H2), lambda i: (0, 0)),
        out_shape=jax.ShapeDtypeStruct((2, H2), _f32),
    )(q[0], q[1], d0, d1)

    out = pl.pallas_call(
        _ntn_body,
        out_shape=jax.ShapeDtypeStruct((1, 1), _f32),
    )(pooled, ntn_W, ntn_V, ntn_b.reshape(K_NTN, 1), u.reshape(K_NTN, 1))
    return out[0, 0]


# speedup vs baseline: 23.3713x; 2.0643x over previous
import functools

import jax
import jax.numpy as jnp
import numpy as np
from jax import lax
from jax.experimental import pallas as pl
from jax.experimental.pallas import tpu as pltpu
from jax.experimental.pallas import tpu_sc as plsc

N = 10000
E = 320000
D_IN = 128
H1 = 64
H2 = 32
K_NTN = 16

NN = 2 * N
NP = 20096
PAD_ROW = NN

NC = 2
NS = 16
NW = NC * NS
CH = 128
CPW = 160
EP = NW * CPW * CH
KSUB = 8
ITERS = CPW // KSUB
RPT = NP // NS
KSUB_BY_D = {64: 4, 32: 8}

_f32 = jnp.float32


def _sc_mesh():
    return plsc.VectorSubcoreMesh(core_axis_name="c", subcore_axis_name="s")


_SC_PARAMS = pltpu.CompilerParams(use_tc_tiling_on_sc=False)


@functools.partial(
    pl.kernel,
    out_type=jax.ShapeDtypeStruct((NC, NP, 16), _f32),
    mesh=_sc_mesh(),
    compiler_params=_SC_PARAMS,
    scratch_types=[
        pltpu.VMEM_SHARED((NP, 16), _f32),
        pltpu.VMEM((KSUB, CH), jnp.int32),
        pltpu.VMEM((CH, 16), _f32),
    ],
)
def _sc_degree(dst_hbm, ones_hbm, zeros_hbm, out_hbm, acc, didx, ones_v):
    c = lax.axis_index("c")
    s = lax.axis_index("s")
    wid = s * NC + c
    pltpu.sync_copy(zeros_hbm.at[pl.ds(s * RPT, RPT)], acc.at[pl.ds(s * RPT, RPT)])
    pltpu.sync_copy(ones_hbm, ones_v)
    plsc.subcore_barrier()

    def body(t, carry):
        row0 = wid * CPW + t * KSUB
        pltpu.sync_copy(dst_hbm.at[pl.ds(row0, KSUB)], didx)
        for j in range(KSUB):
            pltpu.sync_copy(ones_v, acc.at[didx.at[j]], add=True)
        return carry

    lax.fori_loop(0, ITERS, body, 0)
    plsc.subcore_barrier()
    pltpu.sync_copy(acc.at[pl.ds(s * RPT, RPT)], out_hbm.at[c, pl.ds(s * RPT, RPT)])


def _make_segsum():
    ksub = 8
    iters = CPW // ksub
    dh = 32

    @functools.partial(
        pl.kernel,
        out_type=jax.ShapeDtypeStruct((NC, NP, dh), _f32),
        mesh=_sc_mesh(),
        compiler_params=_SC_PARAMS,
        scratch_types=[
            pltpu.VMEM_SHARED((NP, dh), _f32),
            pltpu.VMEM_SHARED((NP, dh), _f32),
            pltpu.VMEM((ksub, CH), jnp.int32),
            pltpu.VMEM((ksub, CH), jnp.int32),
            pltpu.VMEM((ksub, CH, dh), _f32),
            pltpu.SemaphoreType.DMA,
        ],
    )
    def seg(z_hbm, src_hbm, dst_hbm, zeros_hbm, out_hbm, zloc, acc, sidx, didx, rows, sem):
        c = lax.axis_index("c")
        s = lax.axis_index("s")
        wid = s * NC + c
        pltpu.sync_copy(z_hbm.at[pl.ds(s * RPT, RPT)], zloc.at[pl.ds(s * RPT, RPT)])
        pltpu.sync_copy(zeros_hbm.at[pl.ds(s * RPT, RPT)], acc.at[pl.ds(s * RPT, RPT)])
        plsc.subcore_barrier()

        def body(t, carry):
            row0 = wid * CPW + t * ksub
            pltpu.sync_copy(src_hbm.at[pl.ds(row0, ksub)], sidx)
            pltpu.sync_copy(dst_hbm.at[pl.ds(row0, ksub)], didx)
            descs = [
                pltpu.async_copy(zloc.at[sidx.at[j]], rows.at[j], sem)
                for j in range(ksub)
            ]
            for dsc in descs:
                dsc.wait()
            for j in range(ksub):
                pltpu.sync_copy(rows.at[j], acc.at[didx.at[j]], add=True)
            return carry

        lax.fori_loop(0, iters, body, 0)
        plsc.subcore_barrier()
        pltpu.sync_copy(acc.at[pl.ds(s * RPT, RPT)], out_hbm.at[c, pl.ds(s * RPT, RPT)])

    return seg


_segsum_32 = _make_segsum()


BM = 1256


def _r_from_deg(d0, d1):
    deg = d0[:, :1] + d1[:, :1]
    return lax.rsqrt(jnp.maximum(deg, 1.0))


def _mm1_body(x_ref, d0_ref, d1_ref, w_ref, oa_ref, ob_ref):
    r = _r_from_deg(d0_ref[...], d1_ref[...])
    z = r * jnp.dot(x_ref[...], w_ref[...], preferred_element_type=_f32)
    oa_ref[...] = z[:, :H2]
    ob_ref[...] = z[:, H2:]


def _mm2_body(pa0_ref, pa1_ref, pb0_ref, pb1_ref, d0_ref, d1_ref, w_ref, o_ref):
    r = _r_from_deg(d0_ref[...], d1_ref[...])
    agg = jnp.concatenate([pa0_ref[...] + pa1_ref[...],
                           pb0_ref[...] + pb1_ref[...]], axis=1)
    h = jnp.maximum(r * agg, 0.0)
    o_ref[...] = r * jnp.dot(h, w_ref[...], preferred_element_type=_f32)


BP = 1000


def _pool_body(p0_ref, p1_ref, d0_ref, d1_ref, o_ref):
    i = pl.program_id(0)
    r = _r_from_deg(d0_ref[...], d1_ref[...])
    h = jnp.maximum(r * (p0_ref[...] + p1_ref[...]), 0.0)
    colsum = jnp.sum(h, axis=0, keepdims=True) * np.float32(1.0 / N)

    @pl.when(i == 0)
    def _():
        o_ref[...] = jnp.zeros_like(o_ref)

    @pl.when(i < 10)
    def _():
        o_ref[0:1, :] += colsum

    @pl.when(i >= 10)
    def _():
        o_ref[1:2, :] += colsum


def _ntn_body(p_ref, w_ref, v_ref, b_ref, u_ref, o_ref):
    p = p_ref[...]
    h1 = p[0:1, :]
    h2 = p[1:2, :]
    w = w_ref[...]
    t = jnp.sum(w * h2[None, :, :], axis=2)
    bil = jnp.sum(t * h1, axis=1, keepdims=True)
    v = v_ref[...]
    lin = (jnp.sum(v[:, :H2] * h1, axis=1, keepdims=True)
           + jnp.sum(v[:, H2:] * h2, axis=1, keepdims=True))
    scores = jnp.maximum(bil + lin + b_ref[...], 0.0)
    val = jnp.sum(u_ref[...] * scores, keepdims=True)
    o_ref[...] = 1.0 / (1.0 + jnp.exp(-val))


def kernel(features_1, features_2, edge_index_1, edge_index_2,
           W1, W2, ntn_W, ntn_V, ntn_b, u):
    x = jnp.concatenate([features_1, features_2], axis=0)
    x = jnp.pad(x, ((0, NP - NN), (0, 0)))
    src = jnp.concatenate([
        edge_index_1[0], edge_index_2[0] + N,
        jnp.full((EP - 2 * E,), PAD_ROW, jnp.int32),
    ]).reshape(EP // CH, CH)
    dst = jnp.concatenate([
        edge_index_1[1], edge_index_2[1] + N,
        jnp.full((EP - 2 * E,), PAD_ROW, jnp.int32),
    ]).reshape(EP // CH, CH)

    ones16 = jnp.ones((CH, 16), _f32)
    zeros16 = jnp.zeros((NP, 16), _f32)
    zeros32 = jnp.zeros((NP, H2), _f32)

    dpart = _sc_degree(dst, ones16, zeros16)
    d0, d1 = dpart[0], dpart[1]

    grid = NP // BM
    z1a, z1b = pl.pallas_call(
        _mm1_body,
        grid=(grid,),
        in_specs=[
            pl.BlockSpec((BM, D_IN), lambda i: (i, 0)),
            pl.BlockSpec((BM, 16), lambda i: (i, 0)),
            pl.BlockSpec((BM, 16), lambda i: (i, 0)),
            pl.BlockSpec((D_IN, H1), lambda i: (0, 0)),
        ],
        out_specs=[
            pl.BlockSpec((BM, H2), lambda i: (i, 0)),
            pl.BlockSpec((BM, H2), lambda i: (i, 0)),
        ],
        out_shape=[
            jax.ShapeDtypeStruct((NP, H2), _f32),
            jax.ShapeDtypeStruct((NP, H2), _f32),
        ],
    )(x, d0, d1, W1)

    pa = _segsum_32(z1a, src, dst, zeros32)
    pb = _segsum_32(z1b, src, dst, zeros32)

    z2 = pl.pallas_call(
        _mm2_body,
        grid=(grid,),
        in_specs=[
            pl.BlockSpec((BM, H2), lambda i: (i, 0)),
            pl.BlockSpec((BM, H2), lambda i: (i, 0)),
            pl.BlockSpec((BM, H2), lambda i: (i, 0)),
            pl.BlockSpec((BM, H2), lambda i: (i, 0)),
            pl.BlockSpec((BM, 16), lambda i: (i, 0)),
            pl.BlockSpec((BM, 16), lambda i: (i, 0)),
            pl.BlockSpec((H1, H2), lambda i: (0, 0)),
        ],
        out_specs=pl.BlockSpec((BM, H2), lambda i: (i, 0)),
        out_shape=jax.ShapeDtypeStruct((NP, H2), _f32),
    )(pa[0], pa[1], pb[0], pb[1], d0, d1, W2)

    q = _segsum_32(z2, src, dst, zeros32)

    pooled = pl.pallas_call(
        _pool_body,
        grid=(2 * N // BP,),
        in_specs=[
            pl.BlockSpec((BP, H2), lambda i: (i, 0)),
            pl.BlockSpec((BP, H2), lambda i: (i, 0)),
            pl.BlockSpec((BP, 16), lambda i: (i, 0)),
            pl.BlockSpec((BP, 16), lambda i: (i, 0)),
        ],
        out_specs=pl.BlockSpec((2, H2), lambda i: (0, 0)),
        out_shape=jax.ShapeDtypeStruct((2, H2), _f32),
    )(q[0], q[1], d0, d1)

    out = pl.pallas_call(
        _ntn_body,
        out_shape=jax.ShapeDtypeStruct((1, 1), _f32),
    )(pooled, ntn_W, ntn_V, ntn_b.reshape(K_NTN, 1), u.reshape(K_NTN, 1))
    return out[0, 0]


# speedup vs baseline: 26.3146x; 1.1259x over previous
import functools

import jax
import jax.numpy as jnp
import numpy as np
from jax import lax
from jax.experimental import pallas as pl
from jax.experimental.pallas import tpu as pltpu
from jax.experimental.pallas import tpu_sc as plsc

N = 10000
E = 320000
D_IN = 128
H1 = 64
H2 = 32
K_NTN = 16

NN = 2 * N
NP = 20096
PAD_ROW = NN

NC = 2
NS = 16
NW = NC * NS
CH = 128
CPW = 160
EP = NW * CPW * CH
KSUB = 8
RPT = NP // NS

_f32 = jnp.float32


def _sc_mesh():
    return plsc.VectorSubcoreMesh(core_axis_name="c", subcore_axis_name="s")


_SC_PARAMS = pltpu.CompilerParams(use_tc_tiling_on_sc=False)


@functools.partial(
    pl.kernel,
    out_type=(jax.ShapeDtypeStruct((NP, 16), _f32),
              jax.ShapeDtypeStruct((NP, 16), _f32)),
    mesh=_sc_mesh(),
    compiler_params=_SC_PARAMS,
    scratch_types=[
        pltpu.VMEM_SHARED((NP, 16), _f32),
        pltpu.VMEM((KSUB, CH), jnp.int32),
        pltpu.VMEM((CH, 16), _f32),
        pltpu.SemaphoreType.DMA,
    ],
)
def _sc_degree(dst_hbm, ones_hbm, zeros_hbm, out0_hbm, out1_hbm,
               acc, didx, ones_v, sem):
    c = lax.axis_index("c")
    s = lax.axis_index("s")
    wid = s * NC + c
    pltpu.sync_copy(zeros_hbm.at[pl.ds(s * RPT, RPT)], acc.at[pl.ds(s * RPT, RPT)])
    pltpu.sync_copy(ones_hbm, ones_v)
    plsc.subcore_barrier()

    def body(t, carry):
        row0 = wid * CPW + t * KSUB
        pltpu.sync_copy(dst_hbm.at[pl.ds(row0, KSUB)], didx)
        descs = [
            pltpu.async_copy(ones_v, acc.at[didx.at[j]], sem, add=True)
            for j in range(KSUB)
        ]
        for dsc in descs:
            dsc.wait()
        return carry

    lax.fori_loop(0, CPW // KSUB, body, 0)
    plsc.subcore_barrier()

    @pl.when(c == 0)
    def _():
        pltpu.sync_copy(acc.at[pl.ds(s * RPT, RPT)], out0_hbm.at[pl.ds(s * RPT, RPT)])

    @pl.when(c == 1)
    def _():
        pltpu.sync_copy(acc.at[pl.ds(s * RPT, RPT)], out1_hbm.at[pl.ds(s * RPT, RPT)])


def _edge_walk(z_hbm, zeros_hbm, src_hbm, dst_hbm, zloc, acc,
               sidx, didx, rows, semg, sems, s, chunk0, chunks):
    pltpu.sync_copy(z_hbm.at[pl.ds(s * RPT, RPT)], zloc.at[pl.ds(s * RPT, RPT)])
    pltpu.sync_copy(zeros_hbm.at[pl.ds(s * RPT, RPT)], acc.at[pl.ds(s * RPT, RPT)])
    plsc.subcore_barrier()

    def body(t, carry):
        row0 = chunk0 + t * KSUB
        pltpu.sync_copy(src_hbm.at[pl.ds(row0, KSUB)], sidx)
        pltpu.sync_copy(dst_hbm.at[pl.ds(row0, KSUB)], didx)
        gath = [
            pltpu.async_copy(zloc.at[sidx.at[j]], rows.at[j], semg)
            for j in range(KSUB)
        ]
        scat = []
        for j in range(KSUB):
            gath[j].wait()
            scat.append(
                pltpu.async_copy(rows.at[j], acc.at[didx.at[j]], sems, add=True))
        for dsc in scat:
            dsc.wait()
        return carry

    lax.fori_loop(0, chunks // KSUB, body, 0)
    plsc.subcore_barrier()


def _seg_scratch():
    return [
        pltpu.VMEM_SHARED((NP, H2), _f32),
        pltpu.VMEM_SHARED((NP, H2), _f32),
        pltpu.VMEM((KSUB, CH), jnp.int32),
        pltpu.VMEM((KSUB, CH), jnp.int32),
        pltpu.VMEM((KSUB, CH, H2), _f32),
        pltpu.SemaphoreType.DMA,
        pltpu.SemaphoreType.DMA,
    ]


@functools.partial(
    pl.kernel,
    out_type=(jax.ShapeDtypeStruct((NP, H2), _f32),
              jax.ShapeDtypeStruct((NP, H2), _f32)),
    mesh=_sc_mesh(),
    compiler_params=_SC_PARAMS,
    scratch_types=_seg_scratch(),
)
def _seg_l1(za_hbm, zb_hbm, src_hbm, dst_hbm, zeros_hbm, outa_hbm, outb_hbm,
            zloc, acc, sidx, didx, rows, semg, sems):
    c = lax.axis_index("c")
    s = lax.axis_index("s")

    @pl.when(c == 0)
    def _():
        _edge_walk(za_hbm, zeros_hbm, src_hbm, dst_hbm, zloc, acc,
                   sidx, didx, rows, semg, sems, s, s * (2 * CPW), 2 * CPW)
        pltpu.sync_copy(acc.at[pl.ds(s * RPT, RPT)], outa_hbm.at[pl.ds(s * RPT, RPT)])

    @pl.when(c == 1)
    def _():
        _edge_walk(zb_hbm, zeros_hbm, src_hbm, dst_hbm, zloc, acc,
                   sidx, didx, rows, semg, sems, s, s * (2 * CPW), 2 * CPW)
        pltpu.sync_copy(acc.at[pl.ds(s * RPT, RPT)], outb_hbm.at[pl.ds(s * RPT, RPT)])


@functools.partial(
    pl.kernel,
    out_type=(jax.ShapeDtypeStruct((NP, H2), _f32),
              jax.ShapeDtypeStruct((NP, H2), _f32)),
    mesh=_sc_mesh(),
    compiler_params=_SC_PARAMS,
    scratch_types=_seg_scratch(),
)
def _seg_l2(z_hbm, src_hbm, dst_hbm, zeros_hbm, out0_hbm, out1_hbm,
            zloc, acc, sidx, didx, rows, semg, sems):
    c = lax.axis_index("c")
    s = lax.axis_index("s")
    wid = s * NC + c
    _edge_walk(z_hbm, zeros_hbm, src_hbm, dst_hbm, zloc, acc,
               sidx, didx, rows, semg, sems, s, wid * CPW, CPW)

    @pl.when(c == 0)
    def _():
        pltpu.sync_copy(acc.at[pl.ds(s * RPT, RPT)], out0_hbm.at[pl.ds(s * RPT, RPT)])

    @pl.when(c == 1)
    def _():
        pltpu.sync_copy(acc.at[pl.ds(s * RPT, RPT)], out1_hbm.at[pl.ds(s * RPT, RPT)])


BM = 1256


def _r_from_deg(d0, d1):
    deg = d0[:, :1] + d1[:, :1]
    return lax.rsqrt(jnp.maximum(deg, 1.0))


def _mm1_body(x_ref, d0_ref, d1_ref, w_ref, oa_ref, ob_ref):
    r = _r_from_deg(d0_ref[...], d1_ref[...])
    z = r * jnp.dot(x_ref[...], w_ref[...], preferred_element_type=_f32)
    oa_ref[...] = z[:, :H2]
    ob_ref[...] = z[:, H2:]


def _mm2_body(pa_ref, pb_ref, d0_ref, d1_ref, w_ref, o_ref):
    r = _r_from_deg(d0_ref[...], d1_ref[...])
    agg = jnp.concatenate([pa_ref[...], pb_ref[...]], axis=1)
    h = jnp.maximum(r * agg, 0.0)
    o_ref[...] = r * jnp.dot(h, w_ref[...], preferred_element_type=_f32)


BP = 1000


def _pool_ntn_body(q0_ref, q1_ref, d0_ref, d1_ref, w_ref, v_ref, b_ref, u_ref,
                   o_ref, pool_acc):
    i = pl.program_id(0)

    @pl.when(i == 0)
    def _():
        pool_acc[...] = jnp.zeros_like(pool_acc)

    @pl.when(i < 20)
    def _():
        r = _r_from_deg(d0_ref[...], d1_ref[...])
        h = jnp.maximum(r * (q0_ref[...] + q1_ref[...]), 0.0)
        colsum = jnp.sum(h, axis=0, keepdims=True) * np.float32(1.0 / N)

        @pl.when(i < 10)
        def _():
            pool_acc[0:1, :] += colsum

        @pl.when(jnp.logical_and(i >= 10, i < 20))
        def _():
            pool_acc[1:2, :] += colsum

    @pl.when(i == 20)
    def _():
        p = pool_acc[...]
        h1 = p[0:1, :]
        h2 = p[1:2, :]
        w = w_ref[...]
        t = jnp.sum(w * h2[None, :, :], axis=2)
        bil = jnp.sum(t * h1, axis=1, keepdims=True)
        v = v_ref[...]
        lin = (jnp.sum(v[:, :H2] * h1, axis=1, keepdims=True)
               + jnp.sum(v[:, H2:] * h2, axis=1, keepdims=True))
        scores = jnp.maximum(bil + lin + b_ref[...], 0.0)
        val = jnp.sum(u_ref[...] * scores, keepdims=True)
        o_ref[...] = 1.0 / (1.0 + jnp.exp(-val))


def kernel(features_1, features_2, edge_index_1, edge_index_2,
           W1, W2, ntn_W, ntn_V, ntn_b, u):
    x = jnp.concatenate([features_1, features_2], axis=0)
    x = jnp.pad(x, ((0, NP - NN), (0, 0)))
    src = jnp.concatenate([
        edge_index_1[0], edge_index_2[0] + N,
        jnp.full((EP - 2 * E,), PAD_ROW, jnp.int32),
    ]).reshape(EP // CH, CH)
    dst = jnp.concatenate([
        edge_index_1[1], edge_index_2[1] + N,
        jnp.full((EP - 2 * E,), PAD_ROW, jnp.int32),
    ]).reshape(EP // CH, CH)

    ones16 = jnp.ones((CH, 16), _f32)
    zeros16 = jnp.zeros((NP, 16), _f32)
    zeros32 = jnp.zeros((NP, H2), _f32)

    d0, d1 = _sc_degree(dst, ones16, zeros16)

    grid = NP // BM
    z1a, z1b = pl.pallas_call(
        _mm1_body,
        grid=(grid,),
        in_specs=[
            pl.BlockSpec((BM, D_IN), lambda i: (i, 0)),
            pl.BlockSpec((BM, 16), lambda i: (i, 0)),
            pl.BlockSpec((BM, 16), lambda i: (i, 0)),
            pl.BlockSpec((D_IN, H1), lambda i: (0, 0)),
        ],
        out_specs=[
            pl.BlockSpec((BM, H2), lambda i: (i, 0)),
            pl.BlockSpec((BM, H2), lambda i: (i, 0)),
        ],
        out_shape=[
            jax.ShapeDtypeStruct((NP, H2), _f32),
            jax.ShapeDtypeStruct((NP, H2), _f32),
        ],
    )(x, d0, d1, W1)

    pa, pb = _seg_l1(z1a, z1b, src, dst, zeros32)

    z2 = pl.pallas_call(
        _mm2_body,
        grid=(grid,),
        in_specs=[
            pl.BlockSpec((BM, H2), lambda i: (i, 0)),
            pl.BlockSpec((BM, H2), lambda i: (i, 0)),
            pl.BlockSpec((BM, 16), lambda i: (i, 0)),
            pl.BlockSpec((BM, 16), lambda i: (i, 0)),
            pl.BlockSpec((H1, H2), lambda i: (0, 0)),
        ],
        out_specs=pl.BlockSpec((BM, H2), lambda i: (i, 0)),
        out_shape=jax.ShapeDtypeStruct((NP, H2), _f32),
    )(pa, pb, d0, d1, W2)

    q0, q1 = _seg_l2(z2, src, dst, zeros32)

    out = pl.pallas_call(
        _pool_ntn_body,
        grid=(2 * N // BP + 1,),
        in_specs=[
            pl.BlockSpec((BP, H2), lambda i: (jnp.minimum(i, 19), 0)),
            pl.BlockSpec((BP, H2), lambda i: (jnp.minimum(i, 19), 0)),
            pl.BlockSpec((BP, 16), lambda i: (jnp.minimum(i, 19), 0)),
            pl.BlockSpec((BP, 16), lambda i: (jnp.minimum(i, 19), 0)),
            pl.BlockSpec((K_NTN, H2, H2), lambda i: (0, 0, 0)),
            pl.BlockSpec((K_NTN, 2 * H2), lambda i: (0, 0)),
            pl.BlockSpec((K_NTN, 1), lambda i: (0, 0)),
            pl.BlockSpec((K_NTN, 1), lambda i: (0, 0)),
        ],
        out_specs=pl.BlockSpec((1, 1), lambda i: (0, 0)),
        out_shape=jax.ShapeDtypeStruct((1, 1), _f32),
        scratch_shapes=[pltpu.VMEM((2, H2), _f32)],
    )(q0, q1, d0, d1, ntn_W, ntn_V, ntn_b.reshape(K_NTN, 1), u.reshape(K_NTN, 1))
    return out[0, 0]


# speedup vs baseline: 29.2228x; 1.1105x over previous
import functools

import jax
import jax.numpy as jnp
import numpy as np
from jax import lax
from jax.experimental import pallas as pl
from jax.experimental.pallas import tpu as pltpu
from jax.experimental.pallas import tpu_sc as plsc

N = 10000
E = 320000
D_IN = 128
H1 = 64
H2 = 32
K_NTN = 16

NN = 2 * N
NP = 20096
PAD_ROW = NN

NC = 2
NS = 16
NW = NC * NS
CH = 128
CPW = 160
EP = NW * CPW * CH
KSUB = 10
RPT = NP // NS

_f32 = jnp.float32


def _sc_mesh():
    return plsc.VectorSubcoreMesh(core_axis_name="c", subcore_axis_name="s")


_SC_PARAMS = pltpu.CompilerParams(use_tc_tiling_on_sc=False)


@functools.partial(
    pl.kernel,
    out_type=(jax.ShapeDtypeStruct((NP, 16), _f32),
              jax.ShapeDtypeStruct((NP, 16), _f32)),
    mesh=_sc_mesh(),
    compiler_params=_SC_PARAMS,
    scratch_types=[
        pltpu.VMEM_SHARED((NP, 16), _f32),
        pltpu.VMEM((KSUB, CH), jnp.int32),
        pltpu.VMEM((CH, 16), _f32),
        pltpu.SemaphoreType.DMA,
    ],
)
def _sc_degree(dst_hbm, ones_hbm, zeros_hbm, out0_hbm, out1_hbm,
               acc, didx, ones_v, sem):
    c = lax.axis_index("c")
    s = lax.axis_index("s")
    wid = s * NC + c
    pltpu.sync_copy(zeros_hbm.at[pl.ds(s * RPT, RPT)], acc.at[pl.ds(s * RPT, RPT)])
    pltpu.sync_copy(ones_hbm, ones_v)
    plsc.subcore_barrier()

    def body(t, carry):
        row0 = wid * CPW + t * KSUB
        pltpu.sync_copy(dst_hbm.at[pl.ds(row0, KSUB)], didx)
        descs = [
            pltpu.async_copy(ones_v, acc.at[didx.at[j]], sem, add=True)
            for j in range(KSUB)
        ]
        for dsc in descs:
            dsc.wait()
        return carry

    lax.fori_loop(0, CPW // KSUB, body, 0)
    plsc.subcore_barrier()

    @pl.when(c == 0)
    def _():
        pltpu.sync_copy(acc.at[pl.ds(s * RPT, RPT)], out0_hbm.at[pl.ds(s * RPT, RPT)])

    @pl.when(c == 1)
    def _():
        pltpu.sync_copy(acc.at[pl.ds(s * RPT, RPT)], out1_hbm.at[pl.ds(s * RPT, RPT)])


def _edge_walk(z_hbm, zeros_hbm, src_hbm, dst_hbm, zloc, acc,
               sidx, didx, rows, semis, semid, semg, sems, s, chunk0, chunks):
    pltpu.sync_copy(z_hbm.at[pl.ds(s * RPT, RPT)], zloc.at[pl.ds(s * RPT, RPT)])
    pltpu.sync_copy(zeros_hbm.at[pl.ds(s * RPT, RPT)], acc.at[pl.ds(s * RPT, RPT)])
    plsc.subcore_barrier()

    iters = chunks // KSUB
    pltpu.async_copy(src_hbm.at[pl.ds(chunk0, KSUB)], sidx.at[0], semis)
    pltpu.async_copy(dst_hbm.at[pl.ds(chunk0, KSUB)], didx.at[0], semid)

    def body(t, carry):
        b = lax.rem(t, 2)
        pltpu.make_async_copy(src_hbm.at[pl.ds(chunk0, KSUB)], sidx.at[b], semis).wait()
        pltpu.make_async_copy(dst_hbm.at[pl.ds(chunk0, KSUB)], didx.at[b], semid).wait()

        @pl.when(t + 1 < iters)
        def _():
            nxt = chunk0 + (t + 1) * KSUB
            pltpu.async_copy(src_hbm.at[pl.ds(nxt, KSUB)], sidx.at[1 - b], semis)
            pltpu.async_copy(dst_hbm.at[pl.ds(nxt, KSUB)], didx.at[1 - b], semid)

        gath = [
            pltpu.async_copy(zloc.at[sidx.at[b, j]], rows.at[j], semg)
            for j in range(KSUB)
        ]
        scat = []
        for j in range(KSUB):
            gath[j].wait()
            scat.append(
                pltpu.async_copy(rows.at[j], acc.at[didx.at[b, j]], sems, add=True))
        for dsc in scat:
            dsc.wait()
        return carry

    lax.fori_loop(0, iters, body, 0)
    plsc.subcore_barrier()


def _seg_scratch():
    return [
        pltpu.VMEM_SHARED((NP, H2), _f32),
        pltpu.VMEM_SHARED((NP, H2), _f32),
        pltpu.VMEM((2, KSUB, CH), jnp.int32),
        pltpu.VMEM((2, KSUB, CH), jnp.int32),
        pltpu.VMEM((KSUB, CH, H2), _f32),
        pltpu.SemaphoreType.DMA,
        pltpu.SemaphoreType.DMA,
        pltpu.SemaphoreType.DMA,
        pltpu.SemaphoreType.DMA,
    ]


@functools.partial(
    pl.kernel,
    out_type=(jax.ShapeDtypeStruct((NP, H2), _f32),
              jax.ShapeDtypeStruct((NP, H2), _f32)),
    mesh=_sc_mesh(),
    compiler_params=_SC_PARAMS,
    scratch_types=_seg_scratch(),
)
def _seg_l1(za_hbm, zb_hbm, src_hbm, dst_hbm, zeros_hbm, outa_hbm, outb_hbm,
            zloc, acc, sidx, didx, rows, semis, semid, semg, sems):
    c = lax.axis_index("c")
    s = lax.axis_index("s")

    @pl.when(c == 0)
    def _():
        _edge_walk(za_hbm, zeros_hbm, src_hbm, dst_hbm, zloc, acc, sidx, didx,
                   rows, semis, semid, semg, sems, s, s * (2 * CPW), 2 * CPW)
        pltpu.sync_copy(acc.at[pl.ds(s * RPT, RPT)], outa_hbm.at[pl.ds(s * RPT, RPT)])

    @pl.when(c == 1)
    def _():
        _edge_walk(zb_hbm, zeros_hbm, src_hbm, dst_hbm, zloc, acc, sidx, didx,
                   rows, semis, semid, semg, sems, s, s * (2 * CPW), 2 * CPW)
        pltpu.sync_copy(acc.at[pl.ds(s * RPT, RPT)], outb_hbm.at[pl.ds(s * RPT, RPT)])


@functools.partial(
    pl.kernel,
    out_type=(jax.ShapeDtypeStruct((NP, H2), _f32),
              jax.ShapeDtypeStruct((NP, H2), _f32)),
    mesh=_sc_mesh(),
    compiler_params=_SC_PARAMS,
    scratch_types=_seg_scratch(),
)
def _seg_l2(z_hbm, src_hbm, dst_hbm, zeros_hbm, out0_hbm, out1_hbm,
            zloc, acc, sidx, didx, rows, semis, semid, semg, sems):
    c = lax.axis_index("c")
    s = lax.axis_index("s")
    wid = s * NC + c
    _edge_walk(z_hbm, zeros_hbm, src_hbm, dst_hbm, zloc, acc, sidx, didx,
               rows, semis, semid, semg, sems, s, wid * CPW, CPW)

    @pl.when(c == 0)
    def _():
        pltpu.sync_copy(acc.at[pl.ds(s * RPT, RPT)], out0_hbm.at[pl.ds(s * RPT, RPT)])

    @pl.when(c == 1)
    def _():
        pltpu.sync_copy(acc.at[pl.ds(s * RPT, RPT)], out1_hbm.at[pl.ds(s * RPT, RPT)])


BM = 1256


def _r_from_deg(d0, d1):
    deg = d0[:, :1] + d1[:, :1]
    return lax.rsqrt(jnp.maximum(deg, 1.0))


def _mm1_body(x_ref, d0_ref, d1_ref, w_ref, oa_ref, ob_ref):
    r = _r_from_deg(d0_ref[...], d1_ref[...])
    z = r * jnp.dot(x_ref[...], w_ref[...], preferred_element_type=_f32)
    oa_ref[...] = z[:, :H2]
    ob_ref[...] = z[:, H2:]


def _mm2_body(pa_ref, pb_ref, d0_ref, d1_ref, w_ref, o_ref):
    r = _r_from_deg(d0_ref[...], d1_ref[...])
    agg = jnp.concatenate([pa_ref[...], pb_ref[...]], axis=1)
    h = jnp.maximum(r * agg, 0.0)
    o_ref[...] = r * jnp.dot(h, w_ref[...], preferred_element_type=_f32)


BP = 1000


def _pool_ntn_body(q0_ref, q1_ref, d0_ref, d1_ref, w_ref, v_ref, b_ref, u_ref,
                   o_ref, pool_acc):
    i = pl.program_id(0)

    @pl.when(i == 0)
    def _():
        pool_acc[...] = jnp.zeros_like(pool_acc)

    @pl.when(i < 20)
    def _():
        r = _r_from_deg(d0_ref[...], d1_ref[...])
        h = jnp.maximum(r * (q0_ref[...] + q1_ref[...]), 0.0)
        colsum = jnp.sum(h, axis=0, keepdims=True) * np.float32(1.0 / N)

        @pl.when(i < 10)
        def _():
            pool_acc[0:1, :] += colsum

        @pl.when(jnp.logical_and(i >= 10, i < 20))
        def _():
            pool_acc[1:2, :] += colsum

    @pl.when(i == 20)
    def _():
        p = pool_acc[...]
        h1 = p[0:1, :]
        h2 = p[1:2, :]
        w = w_ref[...]
        t = jnp.sum(w * h2[None, :, :], axis=2)
        bil = jnp.sum(t * h1, axis=1, keepdims=True)
        v = v_ref[...]
        lin = (jnp.sum(v[:, :H2] * h1, axis=1, keepdims=True)
               + jnp.sum(v[:, H2:] * h2, axis=1, keepdims=True))
        scores = jnp.maximum(bil + lin + b_ref[...], 0.0)
        val = jnp.sum(u_ref[...] * scores, keepdims=True)
        o_ref[...] = 1.0 / (1.0 + jnp.exp(-val))


def kernel(features_1, features_2, edge_index_1, edge_index_2,
           W1, W2, ntn_W, ntn_V, ntn_b, u):
    x = jnp.concatenate([features_1, features_2], axis=0)
    x = jnp.pad(x, ((0, NP - NN), (0, 0)))
    src = jnp.concatenate([
        edge_index_1[0], edge_index_2[0] + N,
        jnp.full((EP - 2 * E,), PAD_ROW, jnp.int32),
    ]).reshape(EP // CH, CH)
    dst = jnp.concatenate([
        edge_index_1[1], edge_index_2[1] + N,
        jnp.full((EP - 2 * E,), PAD_ROW, jnp.int32),
    ]).reshape(EP // CH, CH)

    ones16 = jnp.ones((CH, 16), _f32)
    zeros16 = jnp.zeros((NP, 16), _f32)
    zeros32 = jnp.zeros((NP, H2), _f32)

    d0, d1 = _sc_degree(dst, ones16, zeros16)

    grid = NP // BM
    z1a, z1b = pl.pallas_call(
        _mm1_body,
        grid=(grid,),
        in_specs=[
            pl.BlockSpec((BM, D_IN), lambda i: (i, 0)),
            pl.BlockSpec((BM, 16), lambda i: (i, 0)),
            pl.BlockSpec((BM, 16), lambda i: (i, 0)),
            pl.BlockSpec((D_IN, H1), lambda i: (0, 0)),
        ],
        out_specs=[
            pl.BlockSpec((BM, H2), lambda i: (i, 0)),
            pl.BlockSpec((BM, H2), lambda i: (i, 0)),
        ],
        out_shape=[
            jax.ShapeDtypeStruct((NP, H2), _f32),
            jax.ShapeDtypeStruct((NP, H2), _f32),
        ],
    )(x, d0, d1, W1)

    pa, pb = _seg_l1(z1a, z1b, src, dst, zeros32)

    z2 = pl.pallas_call(
        _mm2_body,
        grid=(grid,),
        in_specs=[
            pl.BlockSpec((BM, H2), lambda i: (i, 0)),
            pl.BlockSpec((BM, H2), lambda i: (i, 0)),
            pl.BlockSpec((BM, 16), lambda i: (i, 0)),
            pl.BlockSpec((BM, 16), lambda i: (i, 0)),
            pl.BlockSpec((H1, H2), lambda i: (0, 0)),
        ],
        out_specs=pl.BlockSpec((BM, H2), lambda i: (i, 0)),
        out_shape=jax.ShapeDtypeStruct((NP, H2), _f32),
    )(pa, pb, d0, d1, W2)

    q0, q1 = _seg_l2(z2, src, dst, zeros32)

    out = pl.pallas_call(
        _pool_ntn_body,
        grid=(2 * N // BP + 1,),
        in_specs=[
            pl.BlockSpec((BP, H2), lambda i: (jnp.minimum(i, 19), 0)),
            pl.BlockSpec((BP, H2), lambda i: (jnp.minimum(i, 19), 0)),
            pl.BlockSpec((BP, 16), lambda i: (jnp.minimum(i, 19), 0)),
            pl.BlockSpec((BP, 16), lambda i: (jnp.minimum(i, 19), 0)),
            pl.BlockSpec((K_NTN, H2, H2), lambda i: (0, 0, 0)),
            pl.BlockSpec((K_NTN, 2 * H2), lambda i: (0, 0)),
            pl.BlockSpec((K_NTN, 1), lambda i: (0, 0)),
            pl.BlockSpec((K_NTN, 1), lambda i: (0, 0)),
        ],
        out_specs=pl.BlockSpec((1, 1), lambda i: (0, 0)),
        out_shape=jax.ShapeDtypeStruct((1, 1), _f32),
        scratch_shapes=[pltpu.VMEM((2, H2), _f32)],
    )(q0, q1, d0, d1, ntn_W, ntn_V, ntn_b.reshape(K_NTN, 1), u.reshape(K_NTN, 1))
    return out[0, 0]


# speedup vs baseline: 40.3883x; 1.3821x over previous
import functools

import jax
import jax.numpy as jnp
import numpy as np
from jax import lax
from jax.experimental import pallas as pl
from jax.experimental.pallas import tpu as pltpu
from jax.experimental.pallas import tpu_sc as plsc

N = 10000
E = 320000
D_IN = 128
H1 = 64
H2 = 32
K_NTN = 16

NN = 2 * N
NP = 20096
PAD_ROW = NN

NC = 2
NS = 16
NW = NC * NS
CH = 128
CPW = 160
EP = NW * CPW * CH
KSUB = 16
RPT = NP // NS

_f32 = jnp.float32
_bf16 = jnp.bfloat16


def _sc_mesh():
    return plsc.VectorSubcoreMesh(core_axis_name="c", subcore_axis_name="s")


_SC_PARAMS = pltpu.CompilerParams(use_tc_tiling_on_sc=False)


@functools.partial(
    pl.kernel,
    out_type=(jax.ShapeDtypeStruct((NP, 16), _f32),
              jax.ShapeDtypeStruct((NP, 16), _f32)),
    mesh=_sc_mesh(),
    compiler_params=_SC_PARAMS,
    scratch_types=[
        pltpu.VMEM_SHARED((NP, 16), _f32),
        pltpu.VMEM((KSUB, CH), jnp.int32),
        pltpu.VMEM((CH, 16), _f32),
        pltpu.SemaphoreType.DMA,
    ],
)
def _sc_degree(dst_hbm, ones_hbm, zeros_hbm, out0_hbm, out1_hbm,
               acc, didx, ones_v, sem):
    c = lax.axis_index("c")
    s = lax.axis_index("s")
    wid = s * NC + c
    pltpu.sync_copy(zeros_hbm.at[pl.ds(s * RPT, RPT)], acc.at[pl.ds(s * RPT, RPT)])
    pltpu.sync_copy(ones_hbm, ones_v)
    plsc.subcore_barrier()

    def body(t, carry):
        row0 = wid * CPW + t * KSUB
        pltpu.sync_copy(dst_hbm.at[pl.ds(row0, KSUB)], didx)
        descs = [
            pltpu.async_copy(ones_v, acc.at[didx.at[j]], sem, add=True)
            for j in range(KSUB)
        ]
        for dsc in descs:
            dsc.wait()
        return carry

    lax.fori_loop(0, CPW // KSUB, body, 0)
    plsc.subcore_barrier()

    @pl.when(c == 0)
    def _():
        pltpu.sync_copy(acc.at[pl.ds(s * RPT, RPT)], out0_hbm.at[pl.ds(s * RPT, RPT)])

    @pl.when(c == 1)
    def _():
        pltpu.sync_copy(acc.at[pl.ds(s * RPT, RPT)], out1_hbm.at[pl.ds(s * RPT, RPT)])


def _edge_walk(z_hbm, zeros_hbm, src_hbm, dst_hbm, zloc, acc,
               sidx, didx, rows, semis, semid, semg, sems, s, chunk0, chunks):
    pltpu.sync_copy(z_hbm.at[pl.ds(s * RPT, RPT)], zloc.at[pl.ds(s * RPT, RPT)])
    pltpu.sync_copy(zeros_hbm.at[pl.ds(s * RPT, RPT)], acc.at[pl.ds(s * RPT, RPT)])
    plsc.subcore_barrier()

    iters = chunks // KSUB
    pltpu.async_copy(src_hbm.at[pl.ds(chunk0, KSUB)], sidx.at[0], semis)
    pltpu.async_copy(dst_hbm.at[pl.ds(chunk0, KSUB)], didx.at[0], semid)

    def body(t, carry):
        b = lax.rem(t, 2)
        pltpu.make_async_copy(src_hbm.at[pl.ds(chunk0, KSUB)], sidx.at[b], semis).wait()
        pltpu.make_async_copy(dst_hbm.at[pl.ds(chunk0, KSUB)], didx.at[b], semid).wait()

        @pl.when(t + 1 < iters)
        def _():
            nxt = chunk0 + (t + 1) * KSUB
            pltpu.async_copy(src_hbm.at[pl.ds(nxt, KSUB)], sidx.at[1 - b], semis)
            pltpu.async_copy(dst_hbm.at[pl.ds(nxt, KSUB)], didx.at[1 - b], semid)

        gath = [
            pltpu.async_copy(zloc.at[sidx.at[b, j]], rows.at[j], semg)
            for j in range(KSUB)
        ]
        scat = []
        for j in range(KSUB):
            gath[j].wait()
            scat.append(
                pltpu.async_copy(rows.at[j], acc.at[didx.at[b, j]], sems, add=True))
        for dsc in scat:
            dsc.wait()
        return carry

    lax.fori_loop(0, iters, body, 0)
    plsc.subcore_barrier()


def _seg_scratch():
    return [
        pltpu.VMEM_SHARED((NP, H2), _bf16),
        pltpu.VMEM_SHARED((NP, H2), _bf16),
        pltpu.VMEM((2, KSUB, CH), jnp.int32),
        pltpu.VMEM((2, KSUB, CH), jnp.int32),
        pltpu.VMEM((KSUB, CH, H2), _bf16),
        pltpu.SemaphoreType.DMA,
        pltpu.SemaphoreType.DMA,
        pltpu.SemaphoreType.DMA,
        pltpu.SemaphoreType.DMA,
    ]


@functools.partial(
    pl.kernel,
    out_type=(jax.ShapeDtypeStruct((NP, H2), _bf16),
              jax.ShapeDtypeStruct((NP, H2), _bf16)),
    mesh=_sc_mesh(),
    compiler_params=_SC_PARAMS,
    scratch_types=_seg_scratch(),
)
def _seg_l1(za_hbm, zb_hbm, src_hbm, dst_hbm, zeros_hbm, outa_hbm, outb_hbm,
            zloc, acc, sidx, didx, rows, semis, semid, semg, sems):
    c = lax.axis_index("c")
    s = lax.axis_index("s")

    @pl.when(c == 0)
    def _():
        _edge_walk(za_hbm, zeros_hbm, src_hbm, dst_hbm, zloc, acc, sidx, didx,
                   rows, semis, semid, semg, sems, s, s * (2 * CPW), 2 * CPW)
        pltpu.sync_copy(acc.at[pl.ds(s * RPT, RPT)], outa_hbm.at[pl.ds(s * RPT, RPT)])

    @pl.when(c == 1)
    def _():
        _edge_walk(zb_hbm, zeros_hbm, src_hbm, dst_hbm, zloc, acc, sidx, didx,
                   rows, semis, semid, semg, sems, s, s * (2 * CPW), 2 * CPW)
        pltpu.sync_copy(acc.at[pl.ds(s * RPT, RPT)], outb_hbm.at[pl.ds(s * RPT, RPT)])


@functools.partial(
    pl.kernel,
    out_type=(jax.ShapeDtypeStruct((NP, H2), _bf16),
              jax.ShapeDtypeStruct((NP, H2), _bf16)),
    mesh=_sc_mesh(),
    compiler_params=_SC_PARAMS,
    scratch_types=_seg_scratch(),
)
def _seg_l2(z_hbm, src_hbm, dst_hbm, zeros_hbm, out0_hbm, out1_hbm,
            zloc, acc, sidx, didx, rows, semis, semid, semg, sems):
    c = lax.axis_index("c")
    s = lax.axis_index("s")
    wid = s * NC + c
    _edge_walk(z_hbm, zeros_hbm, src_hbm, dst_hbm, zloc, acc, sidx, didx,
               rows, semis, semid, semg, sems, s, wid * CPW, CPW)

    @pl.when(c == 0)
    def _():
        pltpu.sync_copy(acc.at[pl.ds(s * RPT, RPT)], out0_hbm.at[pl.ds(s * RPT, RPT)])

    @pl.when(c == 1)
    def _():
        pltpu.sync_copy(acc.at[pl.ds(s * RPT, RPT)], out1_hbm.at[pl.ds(s * RPT, RPT)])


BM = 2512


def _r_from_deg(d0, d1):
    deg = d0[:, :1] + d1[:, :1]
    return lax.rsqrt(jnp.maximum(deg, 1.0))


def _mm1_body(x_ref, d0_ref, d1_ref, w_ref, oa_ref, ob_ref):
    r = _r_from_deg(d0_ref[...], d1_ref[...])
    z = (r * jnp.dot(x_ref[...], w_ref[...], preferred_element_type=_f32)
         ).astype(_bf16)
    oa_ref[...] = z[:, :H2]
    ob_ref[...] = z[:, H2:]


def _mm2_body(pa_ref, pb_ref, d0_ref, d1_ref, w_ref, o_ref):
    r = _r_from_deg(d0_ref[...], d1_ref[...])
    agg = jnp.concatenate([pa_ref[...], pb_ref[...]], axis=1).astype(_f32)
    h = jnp.maximum(r * agg, 0.0)
    o_ref[...] = (r * jnp.dot(h, w_ref[...], preferred_element_type=_f32)
                  ).astype(_bf16)


BP = 2000


def _pool_ntn_body(q0_ref, q1_ref, d0_ref, d1_ref, w_ref, v_ref, b_ref, u_ref,
                   o_ref, pool_acc):
    i = pl.program_id(0)

    @pl.when(i == 0)
    def _():
        pool_acc[...] = jnp.zeros_like(pool_acc)

    @pl.when(i < 10)
    def _():
        r = _r_from_deg(d0_ref[...], d1_ref[...])
        h = jnp.maximum(
            r * (q0_ref[...].astype(_f32) + q1_ref[...].astype(_f32)), 0.0)
        colsum = jnp.sum(h, axis=0, keepdims=True) * np.float32(1.0 / N)

        @pl.when(i < 5)
        def _():
            pool_acc[0:1, :] += colsum

        @pl.when(jnp.logical_and(i >= 5, i < 10))
        def _():
            pool_acc[1:2, :] += colsum

    @pl.when(i == 10)
    def _():
        p = pool_acc[...]
        h1 = p[0:1, :]
        h2 = p[1:2, :]
        w = w_ref[...]
        t = jnp.sum(w * h2[None, :, :], axis=2)
        bil = jnp.sum(t * h1, axis=1, keepdims=True)
        v = v_ref[...]
        lin = (jnp.sum(v[:, :H2] * h1, axis=1, keepdims=True)
               + jnp.sum(v[:, H2:] * h2, axis=1, keepdims=True))
        scores = jnp.maximum(bil + lin + b_ref[...], 0.0)
        val = jnp.sum(u_ref[...] * scores, keepdims=True)
        o_ref[...] = 1.0 / (1.0 + jnp.exp(-val))


def kernel(features_1, features_2, edge_index_1, edge_index_2,
           W1, W2, ntn_W, ntn_V, ntn_b, u):
    x = jnp.concatenate([features_1, features_2], axis=0)
    x = jnp.pad(x, ((0, NP - NN), (0, 0)))
    src = jnp.concatenate([
        edge_index_1[0], edge_index_2[0] + N,
        jnp.full((EP - 2 * E,), PAD_ROW, jnp.int32),
    ]).reshape(EP // CH, CH)
    dst = jnp.concatenate([
        edge_index_1[1], edge_index_2[1] + N,
        jnp.full((EP - 2 * E,), PAD_ROW, jnp.int32),
    ]).reshape(EP // CH, CH)

    ones16 = jnp.ones((CH, 16), _f32)
    zeros16 = jnp.zeros((NP, 16), _f32)
    zeros32 = jnp.zeros((NP, H2), _bf16)

    d0, d1 = _sc_degree(dst, ones16, zeros16)

    grid = NP // BM
    z1a, z1b = pl.pallas_call(
        _mm1_body,
        grid=(grid,),
        in_specs=[
            pl.BlockSpec((BM, D_IN), lambda i: (i, 0)),
            pl.BlockSpec((BM, 16), lambda i: (i, 0)),
            pl.BlockSpec((BM, 16), lambda i: (i, 0)),
            pl.BlockSpec((D_IN, H1), lambda i: (0, 0)),
        ],
        out_specs=[
            pl.BlockSpec((BM, H2), lambda i: (i, 0)),
            pl.BlockSpec((BM, H2), lambda i: (i, 0)),
        ],
        out_shape=[
            jax.ShapeDtypeStruct((NP, H2), _bf16),
            jax.ShapeDtypeStruct((NP, H2), _bf16),
        ],
    )(x, d0, d1, W1)

    pa, pb = _seg_l1(z1a, z1b, src, dst, zeros32)

    z2 = pl.pallas_call(
        _mm2_body,
        grid=(grid,),
        in_specs=[
            pl.BlockSpec((BM, H2), lambda i: (i, 0)),
            pl.BlockSpec((BM, H2), lambda i: (i, 0)),
            pl.BlockSpec((BM, 16), lambda i: (i, 0)),
            pl.BlockSpec((BM, 16), lambda i: (i, 0)),
            pl.BlockSpec((H1, H2), lambda i: (0, 0)),
        ],
        out_specs=pl.BlockSpec((BM, H2), lambda i: (i, 0)),
        out_shape=jax.ShapeDtypeStruct((NP, H2), _bf16),
    )(pa, pb, d0, d1, W2)

    q0, q1 = _seg_l2(z2, src, dst, zeros32)

    out = pl.pallas_call(
        _pool_ntn_body,
        grid=(2 * N // BP + 1,),
        in_specs=[
            pl.BlockSpec((BP, H2), lambda i: (jnp.minimum(i, 9), 0)),
            pl.BlockSpec((BP, H2), lambda i: (jnp.minimum(i, 9), 0)),
            pl.BlockSpec((BP, 16), lambda i: (jnp.minimum(i, 9), 0)),
            pl.BlockSpec((BP, 16), lambda i: (jnp.minimum(i, 9), 0)),
            pl.BlockSpec((K_NTN, H2, H2), lambda i: (0, 0, 0)),
            pl.BlockSpec((K_NTN, 2 * H2), lambda i: (0, 0)),
            pl.BlockSpec((K_NTN, 1), lambda i: (0, 0)),
            pl.BlockSpec((K_NTN, 1), lambda i: (0, 0)),
        ],
        out_specs=pl.BlockSpec((1, 1), lambda i: (0, 0)),
        out_shape=jax.ShapeDtypeStruct((1, 1), _f32),
        scratch_shapes=[pltpu.VMEM((2, H2), _f32)],
    )(q0, q1, d0, d1, ntn_W, ntn_V, ntn_b.reshape(K_NTN, 1), u.reshape(K_NTN, 1))
    return out[0, 0]


# speedup vs baseline: 43.2923x; 1.0719x over previous
import functools

import jax
import jax.numpy as jnp
import numpy as np
from jax import lax
from jax.experimental import pallas as pl
from jax.experimental.pallas import tpu as pltpu
from jax.experimental.pallas import tpu_sc as plsc

N = 10000
E = 320000
D_IN = 128
H1 = 64
H2 = 32
K_NTN = 16

NN = 2 * N
NP = 20096
PAD_ROW = NN

NC = 2
NS = 16
NW = NC * NS
CH = 128
CPW = 160
EP = NW * CPW * CH
KSUB = 20
RPT = NP // NS

_f32 = jnp.float32
_bf16 = jnp.bfloat16


def _sc_mesh():
    return plsc.VectorSubcoreMesh(core_axis_name="c", subcore_axis_name="s")


_SC_PARAMS = pltpu.CompilerParams(use_tc_tiling_on_sc=False)


@functools.partial(
    pl.kernel,
    out_type=(jax.ShapeDtypeStruct((NP, 16), _bf16),
              jax.ShapeDtypeStruct((NP, 16), _bf16)),
    mesh=_sc_mesh(),
    compiler_params=_SC_PARAMS,
    scratch_types=[
        pltpu.VMEM_SHARED((NP, 16), _bf16),
        pltpu.VMEM((2, KSUB, CH), jnp.int32),
        pltpu.VMEM((CH, 16), _bf16),
        pltpu.SemaphoreType.DMA,
        pltpu.SemaphoreType.DMA,
    ],
)
def _sc_degree(dst_hbm, ones_hbm, zeros_hbm, out0_hbm, out1_hbm,
               acc, didx, ones_v, semid, sem):
    c = lax.axis_index("c")
    s = lax.axis_index("s")
    wid = s * NC + c
    chunk0 = wid * CPW
    iters = CPW // KSUB
    pltpu.async_copy(dst_hbm.at[pl.ds(chunk0, KSUB)], didx.at[0], semid)
    pltpu.sync_copy(zeros_hbm.at[pl.ds(s * RPT, RPT)], acc.at[pl.ds(s * RPT, RPT)])
    pltpu.sync_copy(ones_hbm, ones_v)
    plsc.subcore_barrier()

    def body(t, carry):
        b = lax.rem(t, 2)
        pltpu.make_async_copy(dst_hbm.at[pl.ds(chunk0, KSUB)], didx.at[b], semid).wait()

        @pl.when(t + 1 < iters)
        def _():
            nxt = chunk0 + (t + 1) * KSUB
            pltpu.async_copy(dst_hbm.at[pl.ds(nxt, KSUB)], didx.at[1 - b], semid)

        descs = [
            pltpu.async_copy(ones_v, acc.at[didx.at[b, j]], sem, add=True)
            for j in range(KSUB)
        ]
        for dsc in descs:
            dsc.wait()
        return carry

    lax.fori_loop(0, iters, body, 0)
    plsc.subcore_barrier()

    @pl.when(c == 0)
    def _():
        pltpu.sync_copy(acc.at[pl.ds(s * RPT, RPT)], out0_hbm.at[pl.ds(s * RPT, RPT)])

    @pl.when(c == 1)
    def _():
        pltpu.sync_copy(acc.at[pl.ds(s * RPT, RPT)], out1_hbm.at[pl.ds(s * RPT, RPT)])


def _edge_walk(z_hbm, zeros_hbm, src_hbm, dst_hbm, zloc, acc,
               sidx, didx, rows, semis, semid, semg, sems, s, chunk0, chunks):
    pltpu.sync_copy(z_hbm.at[pl.ds(s * RPT, RPT)], zloc.at[pl.ds(s * RPT, RPT)])
    pltpu.sync_copy(zeros_hbm.at[pl.ds(s * RPT, RPT)], acc.at[pl.ds(s * RPT, RPT)])
    plsc.subcore_barrier()

    iters = chunks // KSUB
    pltpu.async_copy(src_hbm.at[pl.ds(chunk0, KSUB)], sidx.at[0], semis)
    pltpu.async_copy(dst_hbm.at[pl.ds(chunk0, KSUB)], didx.at[0], semid)

    def body(t, carry):
        b = lax.rem(t, 2)
        pltpu.make_async_copy(src_hbm.at[pl.ds(chunk0, KSUB)], sidx.at[b], semis).wait()
        pltpu.make_async_copy(dst_hbm.at[pl.ds(chunk0, KSUB)], didx.at[b], semid).wait()

        @pl.when(t + 1 < iters)
        def _():
            nxt = chunk0 + (t + 1) * KSUB
            pltpu.async_copy(src_hbm.at[pl.ds(nxt, KSUB)], sidx.at[1 - b], semis)
            pltpu.async_copy(dst_hbm.at[pl.ds(nxt, KSUB)], didx.at[1 - b], semid)

        gath = [
            pltpu.async_copy(zloc.at[sidx.at[b, j]], rows.at[j], semg)
            for j in range(KSUB)
        ]
        scat = []
        for j in range(KSUB):
            gath[j].wait()
            scat.append(
                pltpu.async_copy(rows.at[j], acc.at[didx.at[b, j]], sems, add=True))
        for dsc in scat:
            dsc.wait()
        return carry

    lax.fori_loop(0, iters, body, 0)
    plsc.subcore_barrier()


def _seg_scratch():
    return [
        pltpu.VMEM_SHARED((NP, H2), _bf16),
        pltpu.VMEM_SHARED((NP, H2), _bf16),
        pltpu.VMEM((2, KSUB, CH), jnp.int32),
        pltpu.VMEM((2, KSUB, CH), jnp.int32),
        pltpu.VMEM((KSUB, CH, H2), _bf16),
        pltpu.SemaphoreType.DMA,
        pltpu.SemaphoreType.DMA,
        pltpu.SemaphoreType.DMA,
        pltpu.SemaphoreType.DMA,
    ]


@functools.partial(
    pl.kernel,
    out_type=(jax.ShapeDtypeStruct((NP, H2), _bf16),
              jax.ShapeDtypeStruct((NP, H2), _bf16)),
    mesh=_sc_mesh(),
    compiler_params=_SC_PARAMS,
    scratch_types=_seg_scratch(),
)
def _seg_l1(za_hbm, zb_hbm, src_hbm, dst_hbm, zeros_hbm, outa_hbm, outb_hbm,
            zloc, acc, sidx, didx, rows, semis, semid, semg, sems):
    c = lax.axis_index("c")
    s = lax.axis_index("s")

    @pl.when(c == 0)
    def _():
        _edge_walk(za_hbm, zeros_hbm, src_hbm, dst_hbm, zloc, acc, sidx, didx,
                   rows, semis, semid, semg, sems, s, s * (2 * CPW), 2 * CPW)
        pltpu.sync_copy(acc.at[pl.ds(s * RPT, RPT)], outa_hbm.at[pl.ds(s * RPT, RPT)])

    @pl.when(c == 1)
    def _():
        _edge_walk(zb_hbm, zeros_hbm, src_hbm, dst_hbm, zloc, acc, sidx, didx,
                   rows, semis, semid, semg, sems, s, s * (2 * CPW), 2 * CPW)
        pltpu.sync_copy(acc.at[pl.ds(s * RPT, RPT)], outb_hbm.at[pl.ds(s * RPT, RPT)])


@functools.partial(
    pl.kernel,
    out_type=(jax.ShapeDtypeStruct((NP, H2), _bf16),
              jax.ShapeDtypeStruct((NP, H2), _bf16)),
    mesh=_sc_mesh(),
    compiler_params=_SC_PARAMS,
    scratch_types=_seg_scratch(),
)
def _seg_l2(z_hbm, src_hbm, dst_hbm, zeros_hbm, out0_hbm, out1_hbm,
            zloc, acc, sidx, didx, rows, semis, semid, semg, sems):
    c = lax.axis_index("c")
    s = lax.axis_index("s")
    wid = s * NC + c
    _edge_walk(z_hbm, zeros_hbm, src_hbm, dst_hbm, zloc, acc, sidx, didx,
               rows, semis, semid, semg, sems, s, wid * CPW, CPW)

    @pl.when(c == 0)
    def _():
        pltpu.sync_copy(acc.at[pl.ds(s * RPT, RPT)], out0_hbm.at[pl.ds(s * RPT, RPT)])

    @pl.when(c == 1)
    def _():
        pltpu.sync_copy(acc.at[pl.ds(s * RPT, RPT)], out1_hbm.at[pl.ds(s * RPT, RPT)])


BM = 2512


def _mm1_body(x_ref, w_ref, o_ref):
    o_ref[...] = jnp.dot(x_ref[...], w_ref[...],
                         preferred_element_type=_f32).astype(_bf16)


def _scale_body(z_ref, d0_ref, d1_ref, oa_ref, ob_ref, r_ref):
    deg = d0_ref[...][:, :1].astype(_f32) + d1_ref[...][:, :1].astype(_f32)
    r = lax.rsqrt(jnp.maximum(deg, 1.0))
    z = (r * z_ref[...].astype(_f32)).astype(_bf16)
    oa_ref[...] = z[:, :H2]
    ob_ref[...] = z[:, H2:]
    r_ref[...] = r


def _mm2_body(pa_ref, pb_ref, r_ref, w_ref, o_ref):
    r = r_ref[...]
    agg = jnp.concatenate([pa_ref[...], pb_ref[...]], axis=1).astype(_f32)
    h = jnp.maximum(r * agg, 0.0)
    o_ref[...] = (r * jnp.dot(h, w_ref[...], preferred_element_type=_f32)
                  ).astype(_bf16)


BP = 2000


def _pool_ntn_body(q0_ref, q1_ref, r_ref, w_ref, v_ref, b_ref, u_ref,
                   o_ref, pool_acc):
    i = pl.program_id(0)

    @pl.when(i == 0)
    def _():
        pool_acc[...] = jnp.zeros_like(pool_acc)

    @pl.when(i < 10)
    def _():
        r = r_ref[...]
        h = jnp.maximum(
            r * (q0_ref[...].astype(_f32) + q1_ref[...].astype(_f32)), 0.0)
        colsum = jnp.sum(h, axis=0, keepdims=True) * np.float32(1.0 / N)

        @pl.when(i < 5)
        def _():
            pool_acc[0:1, :] += colsum

        @pl.when(jnp.logical_and(i >= 5, i < 10))
        def _():
            pool_acc[1:2, :] += colsum

    @pl.when(i == 10)
    def _():
        p = pool_acc[...]
        h1 = p[0:1, :]
        h2 = p[1:2, :]
        w = w_ref[...]
        t = jnp.sum(w * h2[None, :, :], axis=2)
        bil = jnp.sum(t * h1, axis=1, keepdims=True)
        v = v_ref[...]
        lin = (jnp.sum(v[:, :H2] * h1, axis=1, keepdims=True)
               + jnp.sum(v[:, H2:] * h2, axis=1, keepdims=True))
        scores = jnp.maximum(bil + lin + b_ref[...], 0.0)
        val = jnp.sum(u_ref[...] * scores, keepdims=True)
        o_ref[...] = 1.0 / (1.0 + jnp.exp(-val))


def kernel(features_1, features_2, edge_index_1, edge_index_2,
           W1, W2, ntn_W, ntn_V, ntn_b, u):
    x = jnp.concatenate([features_1, features_2], axis=0)
    x = jnp.pad(x, ((0, NP - NN), (0, 0)))
    src = jnp.concatenate([
        edge_index_1[0], edge_index_2[0] + N,
        jnp.full((EP - 2 * E,), PAD_ROW, jnp.int32),
    ]).reshape(EP // CH, CH)
    dst = jnp.concatenate([
        edge_index_1[1], edge_index_2[1] + N,
        jnp.full((EP - 2 * E,), PAD_ROW, jnp.int32),
    ]).reshape(EP // CH, CH)

    ones16 = jnp.ones((CH, 16), _bf16)
    zeros16 = jnp.zeros((NP, 16), _bf16)
    zeros32 = jnp.zeros((NP, H2), _bf16)

    d0, d1 = _sc_degree(dst, ones16, zeros16)

    grid = NP // BM
    z1raw = pl.pallas_call(
        _mm1_body,
        grid=(grid,),
        in_specs=[
            pl.BlockSpec((BM, D_IN), lambda i: (i, 0)),
            pl.BlockSpec((D_IN, H1), lambda i: (0, 0)),
        ],
        out_specs=pl.BlockSpec((BM, H1), lambda i: (i, 0)),
        out_shape=jax.ShapeDtypeStruct((NP, H1), _bf16),
    )(x, W1)

    z1a, z1b, r = pl.pallas_call(
        _scale_body,
        grid=(grid,),
        in_specs=[
            pl.BlockSpec((BM, H1), lambda i: (i, 0)),
            pl.BlockSpec((BM, 16), lambda i: (i, 0)),
            pl.BlockSpec((BM, 16), lambda i: (i, 0)),
        ],
        out_specs=[
            pl.BlockSpec((BM, H2), lambda i: (i, 0)),
            pl.BlockSpec((BM, H2), lambda i: (i, 0)),
            pl.BlockSpec((BM, 1), lambda i: (i, 0)),
        ],
        out_shape=[
            jax.ShapeDtypeStruct((NP, H2), _bf16),
            jax.ShapeDtypeStruct((NP, H2), _bf16),
            jax.ShapeDtypeStruct((NP, 1), _f32),
        ],
    )(z1raw, d0, d1)

    pa, pb = _seg_l1(z1a, z1b, src, dst, zeros32)

    z2 = pl.pallas_call(
        _mm2_body,
        grid=(grid,),
        in_specs=[
            pl.BlockSpec((BM, H2), lambda i: (i, 0)),
            pl.BlockSpec((BM, H2), lambda i: (i, 0)),
            pl.BlockSpec((BM, 1), lambda i: (i, 0)),
            pl.BlockSpec((H1, H2), lambda i: (0, 0)),
        ],
        out_specs=pl.BlockSpec((BM, H2), lambda i: (i, 0)),
        out_shape=jax.ShapeDtypeStruct((NP, H2), _bf16),
    )(pa, pb, r, W2)

    q0, q1 = _seg_l2(z2, src, dst, zeros32)

    out = pl.pallas_call(
        _pool_ntn_body,
        grid=(2 * N // BP + 1,),
        in_specs=[
            pl.BlockSpec((BP, H2), lambda i: (jnp.minimum(i, 9), 0)),
            pl.BlockSpec((BP, H2), lambda i: (jnp.minimum(i, 9), 0)),
            pl.BlockSpec((BP, 1), lambda i: (jnp.minimum(i, 9), 0)),
            pl.BlockSpec((K_NTN, H2, H2), lambda i: (0, 0, 0)),
            pl.BlockSpec((K_NTN, 2 * H2), lambda i: (0, 0)),
            pl.BlockSpec((K_NTN, 1), lambda i: (0, 0)),
            pl.BlockSpec((K_NTN, 1), lambda i: (0, 0)),
        ],
        out_specs=pl.BlockSpec((1, 1), lambda i: (0, 0)),
        out_shape=jax.ShapeDtypeStruct((1, 1), _f32),
        scratch_shapes=[pltpu.VMEM((2, H2), _f32)],
    )(q0, q1, r, ntn_W, ntn_V, ntn_b.reshape(K_NTN, 1), u.reshape(K_NTN, 1))
    return out[0, 0]


# speedup vs baseline: 50.4675x; 1.1657x over previous
import functools

import jax
import jax.numpy as jnp
import numpy as np
from jax import lax
from jax.experimental import pallas as pl
from jax.experimental.pallas import tpu as pltpu
from jax.experimental.pallas import tpu_sc as plsc

N = 10000
E = 320000
D_IN = 128
H1 = 64
H2 = 32
K_NTN = 16

NPH = 10048
PAD_ROW = N
NP = 2 * NPH

NC = 2
NS = 16
NW = NC * NS
CH = 128
EP = 327680
NCHUNK = EP // CH
KSUB = 20
RPT = NPH // NS

_f32 = jnp.float32
_bf16 = jnp.bfloat16


def _sc_mesh():
    return plsc.VectorSubcoreMesh(core_axis_name="c", subcore_axis_name="s")


_SC_PARAMS = pltpu.CompilerParams(use_tc_tiling_on_sc=False)


@functools.partial(
    pl.kernel,
    out_type=(jax.ShapeDtypeStruct((NPH, 16), _bf16),
              jax.ShapeDtypeStruct((NPH, 16), _bf16)),
    mesh=_sc_mesh(),
    compiler_params=_SC_PARAMS,
    scratch_types=[
        pltpu.VMEM_SHARED((NPH, 16), _bf16),
        pltpu.VMEM((2, KSUB, CH), jnp.int32),
        pltpu.VMEM((CH, 16), _bf16),
        pltpu.SemaphoreType.DMA,
        pltpu.SemaphoreType.DMA,
    ],
)
def _sc_degree(dst_hbm, ones_hbm, zeros_hbm, out0_hbm, out1_hbm,
               acc, didx, ones_v, semid, sem):
    c = lax.axis_index("c")
    s = lax.axis_index("s")
    wid = s * NC + c
    cpw = NCHUNK // NW
    chunk0 = wid * cpw
    iters = cpw // KSUB
    pltpu.async_copy(dst_hbm.at[pl.ds(chunk0, KSUB)], didx.at[0], semid)
    pltpu.sync_copy(zeros_hbm.at[pl.ds(s * RPT, RPT)], acc.at[pl.ds(s * RPT, RPT)])
    pltpu.sync_copy(ones_hbm, ones_v)
    plsc.subcore_barrier()

    def body(t, carry):
        b = lax.rem(t, 2)
        pltpu.make_async_copy(dst_hbm.at[pl.ds(chunk0, KSUB)], didx.at[b], semid).wait()

        @pl.when(t + 1 < iters)
        def _():
            nxt = chunk0 + (t + 1) * KSUB
            pltpu.async_copy(dst_hbm.at[pl.ds(nxt, KSUB)], didx.at[1 - b], semid)

        descs = [
            pltpu.async_copy(ones_v, acc.at[didx.at[b, j]], sem, add=True)
            for j in range(KSUB)
        ]
        for dsc in descs:
            dsc.wait()
        return carry

    lax.fori_loop(0, iters, body, 0)
    plsc.subcore_barrier()

    @pl.when(c == 0)
    def _():
        pltpu.sync_copy(acc.at[pl.ds(s * RPT, RPT)], out0_hbm.at[pl.ds(s * RPT, RPT)])

    @pl.when(c == 1)
    def _():
        pltpu.sync_copy(acc.at[pl.ds(s * RPT, RPT)], out1_hbm.at[pl.ds(s * RPT, RPT)])


def _edge_walk(z_hbm, zeros_hbm, src_hbm, dst_hbm, zloc, acc,
               sidx, didx, rows, semis, semid, semg, sems, s, chunk0, chunks):
    pltpu.async_copy(src_hbm.at[pl.ds(chunk0, KSUB)], sidx.at[0], semis)
    pltpu.async_copy(dst_hbm.at[pl.ds(chunk0, KSUB)], didx.at[0], semid)
    pltpu.sync_copy(z_hbm.at[pl.ds(s * RPT, RPT)], zloc.at[pl.ds(s * RPT, RPT)])
    pltpu.sync_copy(zeros_hbm.at[pl.ds(s * RPT, RPT)], acc.at[pl.ds(s * RPT, RPT)])
    plsc.subcore_barrier()

    iters = chunks // KSUB

    def body(t, carry):
        b = lax.rem(t, 2)
        pltpu.make_async_copy(src_hbm.at[pl.ds(chunk0, KSUB)], sidx.at[b], semis).wait()
        pltpu.make_async_copy(dst_hbm.at[pl.ds(chunk0, KSUB)], didx.at[b], semid).wait()

        @pl.when(t + 1 < iters)
        def _():
            nxt = chunk0 + (t + 1) * KSUB
            pltpu.async_copy(src_hbm.at[pl.ds(nxt, KSUB)], sidx.at[1 - b], semis)
            pltpu.async_copy(dst_hbm.at[pl.ds(nxt, KSUB)], didx.at[1 - b], semid)

        gath = [
            pltpu.async_copy(zloc.at[sidx.at[b, j]], rows.at[j], semg)
            for j in range(KSUB)
        ]
        scat = []
        for j in range(KSUB):
            gath[j].wait()
            scat.append(
                pltpu.async_copy(rows.at[j], acc.at[didx.at[b, j]], sems, add=True))
        for dsc in scat:
            dsc.wait()
        return carry

    lax.fori_loop(0, iters, body, 0)
    plsc.subcore_barrier()


def _seg_scratch():
    return [
        pltpu.VMEM_SHARED((NPH, H2), _bf16),
        pltpu.VMEM_SHARED((NPH, H2), _bf16),
        pltpu.VMEM((2, KSUB, CH), jnp.int32),
        pltpu.VMEM((2, KSUB, CH), jnp.int32),
        pltpu.VMEM((KSUB, CH, H2), _bf16),
        pltpu.SemaphoreType.DMA,
        pltpu.SemaphoreType.DMA,
        pltpu.SemaphoreType.DMA,
        pltpu.SemaphoreType.DMA,
    ]


@functools.partial(
    pl.kernel,
    out_type=(jax.ShapeDtypeStruct((NPH, H2), _bf16),
              jax.ShapeDtypeStruct((NPH, H2), _bf16)),
    mesh=_sc_mesh(),
    compiler_params=_SC_PARAMS,
    scratch_types=_seg_scratch(),
)
def _seg_l1(za_hbm, zb_hbm, src_hbm, dst_hbm, zeros_hbm, outa_hbm, outb_hbm,
            zloc, acc, sidx, didx, rows, semis, semid, semg, sems):
    c = lax.axis_index("c")
    s = lax.axis_index("s")
    cpt = NCHUNK // NS

    @pl.when(c == 0)
    def _():
        _edge_walk(za_hbm, zeros_hbm, src_hbm, dst_hbm, zloc, acc, sidx, didx,
                   rows, semis, semid, semg, sems, s, s * cpt, cpt)
        pltpu.sync_copy(acc.at[pl.ds(s * RPT, RPT)], outa_hbm.at[pl.ds(s * RPT, RPT)])

    @pl.when(c == 1)
    def _():
        _edge_walk(zb_hbm, zeros_hbm, src_hbm, dst_hbm, zloc, acc, sidx, didx,
                   rows, semis, semid, semg, sems, s, s * cpt, cpt)
        pltpu.sync_copy(acc.at[pl.ds(s * RPT, RPT)], outb_hbm.at[pl.ds(s * RPT, RPT)])


@functools.partial(
    pl.kernel,
    out_type=(jax.ShapeDtypeStruct((NPH, H2), _bf16),
              jax.ShapeDtypeStruct((NPH, H2), _bf16)),
    mesh=_sc_mesh(),
    compiler_params=_SC_PARAMS,
    scratch_types=_seg_scratch(),
)
def _seg_l2(z_hbm, src_hbm, dst_hbm, zeros_hbm, out0_hbm, out1_hbm,
            zloc, acc, sidx, didx, rows, semis, semid, semg, sems):
    c = lax.axis_index("c")
    s = lax.axis_index("s")
    wid = s * NC + c
    cpw = NCHUNK // NW
    _edge_walk(z_hbm, zeros_hbm, src_hbm, dst_hbm, zloc, acc, sidx, didx,
               rows, semis, semid, semg, sems, s, wid * cpw, cpw)

    @pl.when(c == 0)
    def _():
        pltpu.sync_copy(acc.at[pl.ds(s * RPT, RPT)], out0_hbm.at[pl.ds(s * RPT, RPT)])

    @pl.when(c == 1)
    def _():
        pltpu.sync_copy(acc.at[pl.ds(s * RPT, RPT)], out1_hbm.at[pl.ds(s * RPT, RPT)])


BM = 2512


def _mm1_body(x_ref, w_ref, o_ref):
    o_ref[...] = jnp.dot(x_ref[...], w_ref[...],
                         preferred_element_type=_f32).astype(_bf16)


def _scale_body(z_ref, d0_ref, d1_ref, oa_ref, ob_ref, r_ref):
    deg = d0_ref[...][:, :1].astype(_f32) + d1_ref[...][:, :1].astype(_f32)
    r = lax.rsqrt(jnp.maximum(deg, 1.0))
    z = (r * z_ref[...].astype(_f32)).astype(_bf16)
    oa_ref[...] = z[:, :H2]
    ob_ref[...] = z[:, H2:]
    r_ref[...] = r


def _mm2_body(pa_ref, pb_ref, r_ref, w_ref, o_ref):
    r = r_ref[...]
    agg = jnp.concatenate([pa_ref[...], pb_ref[...]], axis=1).astype(_f32)
    h = jnp.maximum(r * agg, 0.0)
    o_ref[...] = (r * jnp.dot(h, w_ref[...], preferred_element_type=_f32)
                  ).astype(_bf16)


BP = 2512


def _pool_body(q0_ref, q1_ref, r_ref, o_ref):
    i = pl.program_id(0)
    r = r_ref[...]
    h = jnp.maximum(
        r * (q0_ref[...].astype(_f32) + q1_ref[...].astype(_f32)), 0.0)
    row = i * BP + lax.broadcasted_iota(jnp.int32, (BP, 1), 0)
    h = jnp.where(row < N, h, 0.0)
    colsum = jnp.sum(h, axis=0, keepdims=True) * np.float32(1.0 / N)

    @pl.when(i == 0)
    def _():
        o_ref[...] = jnp.zeros_like(o_ref)

    o_ref[...] += colsum


def _ntn_body(p1_ref, p2_ref, w_ref, v_ref, b_ref, u_ref, o_ref):
    h1 = p1_ref[...]
    h2 = p2_ref[...]
    w = w_ref[...]
    t = jnp.sum(w * h2[None, :, :], axis=2)
    bil = jnp.sum(t * h1, axis=1, keepdims=True)
    v = v_ref[...]
    lin = (jnp.sum(v[:, :H2] * h1, axis=1, keepdims=True)
           + jnp.sum(v[:, H2:] * h2, axis=1, keepdims=True))
    scores = jnp.maximum(bil + lin + b_ref[...], 0.0)
    val = jnp.sum(u_ref[...] * scores, keepdims=True)
    o_ref[...] = 1.0 / (1.0 + jnp.exp(-val))


def _edges2d(ei):
    pad = jnp.full((EP - E,), PAD_ROW, jnp.int32)
    src = jnp.concatenate([ei[0], pad]).reshape(NCHUNK, CH)
    dst = jnp.concatenate([ei[1], pad]).reshape(NCHUNK, CH)
    return src, dst


def kernel(features_1, features_2, edge_index_1, edge_index_2,
           W1, W2, ntn_W, ntn_V, ntn_b, u):
    zpad = jnp.zeros((NPH - N, D_IN), _f32)
    x = jnp.concatenate([features_1, zpad, features_2, zpad])
    edges = [_edges2d(edge_index_1), _edges2d(edge_index_2)]

    ones16 = jnp.ones((CH, 16), _bf16)
    zeros16 = jnp.zeros((NPH, 16), _bf16)
    zeros32 = jnp.zeros((NPH, H2), _bf16)

    degs = [_sc_degree(dst, ones16, zeros16) for (_, dst) in edges]

    z1raw = pl.pallas_call(
        _mm1_body,
        grid=(NP // BM,),
        in_specs=[
            pl.BlockSpec((BM, D_IN), lambda i: (i, 0)),
            pl.BlockSpec((D_IN, H1), lambda i: (0, 0)),
        ],
        out_specs=pl.BlockSpec((BM, H1), lambda i: (i, 0)),
        out_shape=jax.ShapeDtypeStruct((NP, H1), _bf16),
    )(x, W1)

    pooled = []
    for g in (0, 1):
        src, dst = edges[g]
        d0, d1 = degs[g]
        goff = g * (NPH // BM)

        z1a, z1b, r = pl.pallas_call(
            _scale_body,
            grid=(NPH // BM,),
            in_specs=[
                pl.BlockSpec((BM, H1), lambda i, goff=goff: (i + goff, 0)),
                pl.BlockSpec((BM, 16), lambda i: (i, 0)),
                pl.BlockSpec((BM, 16), lambda i: (i, 0)),
            ],
            out_specs=[
                pl.BlockSpec((BM, H2), lambda i: (i, 0)),
                pl.BlockSpec((BM, H2), lambda i: (i, 0)),
                pl.BlockSpec((BM, 1), lambda i: (i, 0)),
            ],
            out_shape=[
                jax.ShapeDtypeStruct((NPH, H2), _bf16),
                jax.ShapeDtypeStruct((NPH, H2), _bf16),
                jax.ShapeDtypeStruct((NPH, 1), _f32),
            ],
        )(z1raw, d0, d1)

        pa, pb = _seg_l1(z1a, z1b, src, dst, zeros32)

        z2 = pl.pallas_call(
            _mm2_body,
            grid=(NPH // BM,),
            in_specs=[
                pl.BlockSpec((BM, H2), lambda i: (i, 0)),
                pl.BlockSpec((BM, H2), lambda i: (i, 0)),
                pl.BlockSpec((BM, 1), lambda i: (i, 0)),
                pl.BlockSpec((H1, H2), lambda i: (0, 0)),
            ],
            out_specs=pl.BlockSpec((BM, H2), lambda i: (i, 0)),
            out_shape=jax.ShapeDtypeStruct((NPH, H2), _bf16),
        )(pa, pb, r, W2)

        q0, q1 = _seg_l2(z2, src, dst, zeros32)

        pooled.append(pl.pallas_call(
            _pool_body,
            grid=(NPH // BP,),
            in_specs=[
                pl.BlockSpec((BP, H2), lambda i: (i, 0)),
                pl.BlockSpec((BP, H2), lambda i: (i, 0)),
                pl.BlockSpec((BP, 1), lambda i: (i, 0)),
            ],
            out_specs=pl.BlockSpec((1, H2), lambda i: (0, 0)),
            out_shape=jax.ShapeDtypeStruct((1, H2), _f32),
        )(q0, q1, r))

    out = pl.pallas_call(
        _ntn_body,
        out_shape=jax.ShapeDtypeStruct((1, 1), _f32),
    )(pooled[0], pooled[1], ntn_W, ntn_V,
      ntn_b.reshape(K_NTN, 1), u.reshape(K_NTN, 1))
    return out[0, 0]


# speedup vs baseline: 54.1378x; 1.0727x over previous
import functools

import jax
import jax.numpy as jnp
import numpy as np
from jax import lax
from jax.experimental import pallas as pl
from jax.experimental.pallas import tpu as pltpu
from jax.experimental.pallas import tpu_sc as plsc

N = 10000
E = 320000
D_IN = 128
H1 = 64
H2 = 32
K_NTN = 16

NPH = 10048

NC = 2
NS = 16
NW = NC * NS
CH = 80
NCHUNK = E // CH
KSUB = 25
RPT = NPH // NS

_f32 = jnp.float32
_bf16 = jnp.bfloat16


def _sc_mesh():
    return plsc.VectorSubcoreMesh(core_axis_name="c", subcore_axis_name="s")


_SC_PARAMS = pltpu.CompilerParams(use_tc_tiling_on_sc=False)


@functools.partial(
    pl.kernel,
    out_type=(jax.ShapeDtypeStruct((NPH, 16), _bf16),
              jax.ShapeDtypeStruct((NPH, 16), _bf16)),
    mesh=_sc_mesh(),
    compiler_params=_SC_PARAMS,
    scratch_types=[
        pltpu.VMEM_SHARED((NPH, 16), _bf16),
        pltpu.VMEM((2, KSUB, CH), jnp.int32),
        pltpu.VMEM((CH, 16), _bf16),
        pltpu.SemaphoreType.DMA,
        pltpu.SemaphoreType.DMA,
    ],
)
def _sc_degree(dst_hbm, ones_hbm, zeros_hbm, out0_hbm, out1_hbm,
               acc, didx, ones_v, semid, sem):
    c = lax.axis_index("c")
    s = lax.axis_index("s")
    wid = s * NC + c
    cpw = NCHUNK // NW
    chunk0 = wid * cpw
    iters = cpw // KSUB
    pltpu.async_copy(dst_hbm.at[pl.ds(chunk0, KSUB)], didx.at[0], semid)
    pltpu.sync_copy(zeros_hbm.at[pl.ds(s * RPT, RPT)], acc.at[pl.ds(s * RPT, RPT)])
    pltpu.sync_copy(ones_hbm, ones_v)
    plsc.subcore_barrier()

    def body(t, carry):
        b = lax.rem(t, 2)
        pltpu.make_async_copy(dst_hbm.at[pl.ds(chunk0, KSUB)], didx.at[b], semid).wait()

        @pl.when(t + 1 < iters)
        def _():
            nxt = chunk0 + (t + 1) * KSUB
            pltpu.async_copy(dst_hbm.at[pl.ds(nxt, KSUB)], didx.at[1 - b], semid)

        descs = [
            pltpu.async_copy(ones_v, acc.at[didx.at[b, j]], sem, add=True)
            for j in range(KSUB)
        ]
        for dsc in descs:
            dsc.wait()
        return carry

    lax.fori_loop(0, iters, body, 0)
    plsc.subcore_barrier()

    @pl.when(c == 0)
    def _():
        pltpu.sync_copy(acc.at[pl.ds(s * RPT, RPT)], out0_hbm.at[pl.ds(s * RPT, RPT)])

    @pl.when(c == 1)
    def _():
        pltpu.sync_copy(acc.at[pl.ds(s * RPT, RPT)], out1_hbm.at[pl.ds(s * RPT, RPT)])


def _edge_walk(z_hbm, zeros_hbm, src_hbm, dst_hbm, zloc, acc,
               sidx, didx, rows, semis, semid, semg, sems, s, chunk0, chunks):
    pltpu.async_copy(src_hbm.at[pl.ds(chunk0, KSUB)], sidx.at[0], semis)
    pltpu.async_copy(dst_hbm.at[pl.ds(chunk0, KSUB)], didx.at[0], semid)
    pltpu.sync_copy(z_hbm.at[pl.ds(s * RPT, RPT)], zloc.at[pl.ds(s * RPT, RPT)])
    pltpu.sync_copy(zeros_hbm.at[pl.ds(s * RPT, RPT)], acc.at[pl.ds(s * RPT, RPT)])
    plsc.subcore_barrier()

    iters = chunks // KSUB

    def body(t, carry):
        b = lax.rem(t, 2)
        pltpu.make_async_copy(src_hbm.at[pl.ds(chunk0, KSUB)], sidx.at[b], semis).wait()
        pltpu.make_async_copy(dst_hbm.at[pl.ds(chunk0, KSUB)], didx.at[b], semid).wait()

        @pl.when(t + 1 < iters)
        def _():
            nxt = chunk0 + (t + 1) * KSUB
            pltpu.async_copy(src_hbm.at[pl.ds(nxt, KSUB)], sidx.at[1 - b], semis)
            pltpu.async_copy(dst_hbm.at[pl.ds(nxt, KSUB)], didx.at[1 - b], semid)

        gath = [
            pltpu.async_copy(zloc.at[sidx.at[b, j]], rows.at[j], semg)
            for j in range(KSUB)
        ]
        scat = []
        for j in range(KSUB):
            gath[j].wait()
            scat.append(
                pltpu.async_copy(rows.at[j], acc.at[didx.at[b, j]], sems, add=True))
        for dsc in scat:
            dsc.wait()
        return carry

    lax.fori_loop(0, iters, body, 0)
    plsc.subcore_barrier()


def _seg_scratch():
    return [
        pltpu.VMEM_SHARED((NPH, H2), _bf16),
        pltpu.VMEM_SHARED((NPH, H2), _bf16),
        pltpu.VMEM((2, KSUB, CH), jnp.int32),
        pltpu.VMEM((2, KSUB, CH), jnp.int32),
        pltpu.VMEM((KSUB, CH, H2), _bf16),
        pltpu.SemaphoreType.DMA,
        pltpu.SemaphoreType.DMA,
        pltpu.SemaphoreType.DMA,
        pltpu.SemaphoreType.DMA,
    ]


@functools.partial(
    pl.kernel,
    out_type=(jax.ShapeDtypeStruct((NPH, H2), _bf16),
              jax.ShapeDtypeStruct((NPH, H2), _bf16)),
    mesh=_sc_mesh(),
    compiler_params=_SC_PARAMS,
    scratch_types=_seg_scratch(),
)
def _seg_l1(za_hbm, zb_hbm, src_hbm, dst_hbm, zeros_hbm, outa_hbm, outb_hbm,
            zloc, acc, sidx, didx, rows, semis, semid, semg, sems):
    c = lax.axis_index("c")
    s = lax.axis_index("s")
    cpt = NCHUNK // NS

    @pl.when(c == 0)
    def _():
        _edge_walk(za_hbm, zeros_hbm, src_hbm, dst_hbm, zloc, acc, sidx, didx,
                   rows, semis, semid, semg, sems, s, s * cpt, cpt)
        pltpu.sync_copy(acc.at[pl.ds(s * RPT, RPT)], outa_hbm.at[pl.ds(s * RPT, RPT)])

    @pl.when(c == 1)
    def _():
        _edge_walk(zb_hbm, zeros_hbm, src_hbm, dst_hbm, zloc, acc, sidx, didx,
                   rows, semis, semid, semg, sems, s, s * cpt, cpt)
        pltpu.sync_copy(acc.at[pl.ds(s * RPT, RPT)], outb_hbm.at[pl.ds(s * RPT, RPT)])


@functools.partial(
    pl.kernel,
    out_type=(jax.ShapeDtypeStruct((NPH, H2), _bf16),
              jax.ShapeDtypeStruct((NPH, H2), _bf16)),
    mesh=_sc_mesh(),
    compiler_params=_SC_PARAMS,
    scratch_types=_seg_scratch(),
)
def _seg_l2(z_hbm, src_hbm, dst_hbm, zeros_hbm, out0_hbm, out1_hbm,
            zloc, acc, sidx, didx, rows, semis, semid, semg, sems):
    c = lax.axis_index("c")
    s = lax.axis_index("s")
    wid = s * NC + c
    cpw = NCHUNK // NW
    _edge_walk(z_hbm, zeros_hbm, src_hbm, dst_hbm, zloc, acc, sidx, didx,
               rows, semis, semid, semg, sems, s, wid * cpw, cpw)

    @pl.when(c == 0)
    def _():
        pltpu.sync_copy(acc.at[pl.ds(s * RPT, RPT)], out0_hbm.at[pl.ds(s * RPT, RPT)])

    @pl.when(c == 1)
    def _():
        pltpu.sync_copy(acc.at[pl.ds(s * RPT, RPT)], out1_hbm.at[pl.ds(s * RPT, RPT)])


BM = 2000


def _mm1_body(x_ref, w_ref, o_ref):
    o_ref[...] = jnp.dot(x_ref[...], w_ref[...],
                         preferred_element_type=_f32).astype(_bf16)


def _scale_body(z_ref, d0_ref, d1_ref, oa_ref, ob_ref, r_ref):
    deg = d0_ref[...][:, :1].astype(_f32) + d1_ref[...][:, :1].astype(_f32)
    r = lax.rsqrt(jnp.maximum(deg, 1.0))
    z = (r * z_ref[...].astype(_f32)).astype(_bf16)
    oa_ref[...] = z[:, :H2]
    ob_ref[...] = z[:, H2:]
    r_ref[...] = r


def _mm2_body(pa_ref, pb_ref, r_ref, w_ref, o_ref):
    r = r_ref[...]
    agg = jnp.concatenate([pa_ref[...], pb_ref[...]], axis=1).astype(_f32)
    h = jnp.maximum(r * agg, 0.0)
    o_ref[...] = (r * jnp.dot(h, w_ref[...], preferred_element_type=_f32)
                  ).astype(_bf16)


BP = 2000


def _pool_body(q0_ref, q1_ref, r_ref, o_ref):
    i = pl.program_id(0)
    r = r_ref[...]
    h = jnp.maximum(
        r * (q0_ref[...].astype(_f32) + q1_ref[...].astype(_f32)), 0.0)
    colsum = jnp.sum(h, axis=0, keepdims=True) * np.float32(1.0 / N)

    @pl.when(i == 0)
    def _():
        o_ref[...] = jnp.zeros_like(o_ref)

    o_ref[...] += colsum


def _ntn_body(p1_ref, p2_ref, w_ref, v_ref, b_ref, u_ref, o_ref):
    h1 = p1_ref[...]
    h2 = p2_ref[...]
    w = w_ref[...]
    t = jnp.sum(w * h2[None, :, :], axis=2)
    bil = jnp.sum(t * h1, axis=1, keepdims=True)
    v = v_ref[...]
    lin = (jnp.sum(v[:, :H2] * h1, axis=1, keepdims=True)
           + jnp.sum(v[:, H2:] * h2, axis=1, keepdims=True))
    scores = jnp.maximum(bil + lin + b_ref[...], 0.0)
    val = jnp.sum(u_ref[...] * scores, keepdims=True)
    o_ref[...] = 1.0 / (1.0 + jnp.exp(-val))


def _edges2d(ei):
    return ei[0].reshape(NCHUNK, CH), ei[1].reshape(NCHUNK, CH)


def kernel(features_1, features_2, edge_index_1, edge_index_2,
           W1, W2, ntn_W, ntn_V, ntn_b, u):
    edges = [_edges2d(edge_index_1), _edges2d(edge_index_2)]
    feats = [features_1, features_2]

    ones16 = jnp.ones((CH, 16), _bf16)
    zeros16 = jnp.zeros((NPH, 16), _bf16)
    zeros32 = jnp.zeros((NPH, H2), _bf16)

    degs = [_sc_degree(dst, ones16, zeros16) for (_, dst) in edges]

    z1raws = [pl.pallas_call(
        _mm1_body,
        grid=(N // BM,),
        in_specs=[
            pl.BlockSpec((BM, D_IN), lambda i: (i, 0)),
            pl.BlockSpec((D_IN, H1), lambda i: (0, 0)),
        ],
        out_specs=pl.BlockSpec((BM, H1), lambda i: (i, 0)),
        out_shape=jax.ShapeDtypeStruct((N, H1), _bf16),
    )(xg, W1) for xg in feats]

    pooled = []
    for g in (0, 1):
        src, dst = edges[g]
        d0, d1 = degs[g]

        z1a, z1b, r = pl.pallas_call(
            _scale_body,
            grid=(N // BM,),
            in_specs=[
                pl.BlockSpec((BM, H1), lambda i: (i, 0)),
                pl.BlockSpec((BM, 16), lambda i: (i, 0)),
                pl.BlockSpec((BM, 16), lambda i: (i, 0)),
            ],
            out_specs=[
                pl.BlockSpec((BM, H2), lambda i: (i, 0)),
                pl.BlockSpec((BM, H2), lambda i: (i, 0)),
                pl.BlockSpec((BM, 1), lambda i: (i, 0)),
            ],
            out_shape=[
                jax.ShapeDtypeStruct((NPH, H2), _bf16),
                jax.ShapeDtypeStruct((NPH, H2), _bf16),
                jax.ShapeDtypeStruct((NPH, 1), _f32),
            ],
        )(z1raws[g], d0, d1)

        pa, pb = _seg_l1(z1a, z1b, src, dst, zeros32)

        z2 = pl.pallas_call(
            _mm2_body,
            grid=(N // BM,),
            in_specs=[
                pl.BlockSpec((BM, H2), lambda i: (i, 0)),
                pl.BlockSpec((BM, H2), lambda i: (i, 0)),
                pl.BlockSpec((BM, 1), lambda i: (i, 0)),
                pl.BlockSpec((H1, H2), lambda i: (0, 0)),
            ],
            out_specs=pl.BlockSpec((BM, H2), lambda i: (i, 0)),
            out_shape=jax.ShapeDtypeStruct((NPH, H2), _bf16),
        )(pa, pb, r, W2)

        q0, q1 = _seg_l2(z2, src, dst, zeros32)

        pooled.append(pl.pallas_call(
            _pool_body,
            grid=(N // BP,),
            in_specs=[
                pl.BlockSpec((BP, H2), lambda i: (i, 0)),
                pl.BlockSpec((BP, H2), lambda i: (i, 0)),
                pl.BlockSpec((BP, 1), lambda i: (i, 0)),
            ],
            out_specs=pl.BlockSpec((1, H2), lambda i: (0, 0)),
            out_shape=jax.ShapeDtypeStruct((1, H2), _f32),
        )(q0, q1, r))

    out = pl.pallas_call(
        _ntn_body,
        out_shape=jax.ShapeDtypeStruct((1, 1), _f32),
    )(pooled[0], pooled[1], ntn_W, ntn_V,
      ntn_b.reshape(K_NTN, 1), u.reshape(K_NTN, 1))
    return out[0, 0]
